# Initial kernel scaffold; baseline (speedup 1.0000x reference)
import jax
import jax.numpy as jnp
from jax.experimental import pallas as pl


def kernel(node_attrs, vectors, senders, receivers, W_e1, W_e2, W_e3, W_e4, W_w0, W_env_0, W_lat1_0, W_lat2_0, W_l1_0, W_l2_0, W_env_1, W_lat1_1, W_lat2_1, W_l1_1, W_l2_1, W_f, W_out):
    raise NotImplementedError("write your pallas kernel here")


# speedup vs baseline: 8.7007x; 8.7007x over previous
import functools

import numpy as np
import jax
import jax.numpy as jnp
from jax import lax
from jax.experimental import pallas as pl
from jax.experimental.pallas import tpu as pltpu
from jax.experimental.pallas import tpu_sc as plsc

N_NODES = 10000
N_EDGES = 160000
D_FEAT = 128
N_BASIS = 8
C = 32
P = 6
RC = 4.0
ANN = 32.0
H = 128

EP = 163840
BE = 512
N_BLK = EP // BE
PAD = EP - N_EDGES

_trapz = getattr(np, 'trapezoid', None) or np.trapz


def _bessel_np(r, n):
    k = np.arange(1, n + 1)[None, :]
    r = r[:, None]
    safe = np.where(r == 0.0, 1.0, r)
    return np.sqrt(2.0) * np.where(r == 0.0, k * np.pi, np.sin(k * np.pi * safe) / safe)


_r = np.linspace(0.0, 1.0, 1000)
_b = _bessel_np(_r, N_BASIS)
_MU = _trapz(_b, _r, axis=0)
_SIG = _trapz((_b - _MU) ** 2, _r, axis=0) ** 0.5
MU_NP = np.asarray(_MU, dtype=np.float32)[None, :]
INVSIG_NP = np.asarray(1.0 / _SIG, dtype=np.float32)[None, :]

SQRT2 = float(np.sqrt(2.0))
SQRT3 = float(np.sqrt(3.0))
SQRT5 = float(np.sqrt(5.0))
SQRT15 = float(np.sqrt(15.0))
INV_SQRT3 = float(1.0 / np.sqrt(3.0))
INV_SQRT5 = float(1.0 / np.sqrt(5.0))
INV_SQRT_ANN = float(1.0 / np.sqrt(ANN))
INV_SQRT125 = float(1.0 / np.sqrt(1.25))
S2 = float(1.0 / np.sqrt(2.0))
S6 = float(1.0 / np.sqrt(6.0))

_F32 = jnp.float32


def _silu(x):
    return x * jax.nn.sigmoid(x)


def _dot(a, b):
    return jnp.dot(a, b, preferred_element_type=jnp.float32)


def _node_proj_body(na_ref, w1b_ref, w1c_ref, ps_ref, pr_ref):
    na = na_ref[...]
    ps_ref[...] = _dot(na, w1b_ref[...])
    pr_ref[...] = _dot(na, w1c_ref[...])


def _pass1_body(vec_ref, gs_ref, gr_ref, w1a_ref, we2_ref, we3_ref, we4_ref,
                wenv_ref, x1_ref, wy_ref, geom_ref):
    v = vec_ref[...] * (1.0 / RC)
    d2 = jnp.sum(v * v, axis=1, keepdims=True)
    d = jnp.sqrt(d2)
    iszero = d == 0.0
    safe = jnp.where(iszero, 1.0, d)
    kpi = np.float32(np.pi) * (
        lax.broadcasted_iota(jnp.int32, (1, N_BASIS), 1) + 1).astype(
            jnp.float32)
    rb = SQRT2 * jnp.where(iszero, kpi, jnp.sin(kpi * safe) / safe)
    rb9 = jnp.concatenate([rb, jnp.ones_like(d)], axis=1)
    pre1 = _dot(rb9, w1a_ref[...]) + gs_ref[...] + gr_ref[...]
    pre1 = jnp.where(iszero, 0.0, pre1)
    x = _silu(pre1)
    x = _silu(_dot(x, we2_ref[...]))
    x = _silu(_dot(x, we3_ref[...]))
    x = _dot(x, we4_ref[...])
    d6 = d2 * d2 * d2
    d7 = d6 * d
    d8 = d7 * d
    env = jnp.where(d < 1.0, 1.0 - 28.0 * d6 + 48.0 * d7 - 21.0 * d8, 0.0)
    x = env * x
    rh = v / safe
    xx = rh[:, 0:1]
    yy = rh[:, 1:2]
    zz = rh[:, 2:3]
    Y = jnp.concatenate([
        jnp.ones_like(xx), SQRT3 * xx, SQRT3 * yy, SQRT3 * zz,
        SQRT15 * xx * yy, SQRT15 * yy * zz, (SQRT5 / 2.0) * (3.0 * zz * zz - 1.0),
        SQRT15 * xx * zz, (SQRT15 / 2.0) * (xx * xx - yy * yy)
    ], axis=1)
    x1_ref[...] = x
    we = _dot(x, wenv_ref[...])
    wy_ref[...] = jnp.concatenate([we * Y[:, j:j + 1] for j in range(9)], axis=1)
    geom_ref[...] = jnp.concatenate(
        [env, Y, jnp.zeros_like(Y[:, :6])], axis=1)


def _pass2_body(x1_ref, a_ref, geom_ref, ww0_ref, wlat1_ref, wlat2_ref,
                wl1_ref, wl2_ref, wenv1_ref, x2_ref, wy_ref, vvt_ref):
    x1 = x1_ref[...]
    g = geom_ref[...]
    env = g[:, 0:1]
    w = _dot(x1, ww0_ref[...])
    s = w[:, 0:C]
    wv = w[:, C:2 * C]
    wt = w[:, 2 * C:3 * C]
    vvg = [g[:, 2 + k:3 + k] * wv for k in range(3)]
    tg = [g[:, 5 + m:6 + m] * wt for m in range(5)]
    A = a_ref[...] * INV_SQRT_ANN
    a0 = A[:, 0:C]
    a1 = [A[:, C * (1 + k):C * (2 + k)] for k in range(3)]
    a2 = [A[:, C * (4 + m):C * (5 + m)] for m in range(5)]
    p00 = a0 * s
    p11 = (a1[0] * vvg[0] + a1[1] * vvg[1] + a1[2] * vvg[2]) * INV_SQRT3
    p22 = (a2[0] * tg[0] + a2[1] * tg[1] + a2[2] * tg[2] + a2[3] * tg[3]
           + a2[4] * tg[4]) * INV_SQRT5
    xl = jnp.concatenate([x1, p00, p11, p22], axis=1)
    y = _dot(_silu(_dot(xl, wlat1_ref[...])), wlat2_ref[...])
    y = env * y
    x2 = (x1 + 0.5 * y) * INV_SQRT125
    x2_ref[...] = x2
    wl1 = wl1_ref[...]
    vv2 = [_dot(jnp.concatenate([a0 * vvg[k], a1[k] * s], axis=1), wl1)
           for k in range(3)]
    ax, ay, az = a1
    bx, by, bz = vvg
    r11 = [S2 * (ax * by + ay * bx),
           S2 * (ay * bz + az * by),
           S6 * (2.0 * az * bz - ax * bx - ay * by),
           S2 * (ax * bz + az * bx),
           S2 * (ax * bx - ay * by)]
    wl2 = wl2_ref[...]
    t2 = [_dot(jnp.concatenate([a0 * tg[m], a2[m] * s, r11[m]], axis=1), wl2)
          for m in range(5)]
    vvt_ref[...] = jnp.concatenate(vv2 + t2, axis=1)
    we = _dot(x2, wenv1_ref[...])
    wy_ref[...] = jnp.concatenate([we * g[:, 1 + j:2 + j] for j in range(9)],
                                  axis=1)


def _pass3_body(x2_ref, a_ref, vvt_ref, geom_ref, wlat1_ref, wlat2_ref,
                wf_ref, wout_ref, o_ref):
    x2 = x2_ref[...]
    env = geom_ref[:, 0:1]
    A = a_ref[...] * INV_SQRT_ANN
    a1 = [A[:, C * (1 + k):C * (2 + k)] for k in range(3)]
    a2 = [A[:, C * (4 + m):C * (5 + m)] for m in range(5)]
    vvt = vvt_ref[...]
    vvg = [vvt[:, C * k:C * (k + 1)] for k in range(3)]
    tg = [vvt[:, 3 * C + C * m:3 * C + C * (m + 1)] for m in range(5)]
    p11 = (a1[0] * vvg[0] + a1[1] * vvg[1] + a1[2] * vvg[2]) * INV_SQRT3
    p22 = (a2[0] * tg[0] + a2[1] * tg[1] + a2[2] * tg[2] + a2[3] * tg[3]
           + a2[4] * tg[4]) * INV_SQRT5
    xl = jnp.concatenate([x2, jnp.zeros_like(p11), p11, p22], axis=1)
    y = _dot(_silu(_dot(xl, wlat1_ref[...])), wlat2_ref[...]) * env
    x3 = (x2 + 0.5 * y) * INV_SQRT125
    xf = _dot(x3, wf_ref[...])
    o_ref[...] = jnp.sum(xf * wout_ref[...], axis=1, keepdims=True)


def _full(shape):
    return pl.BlockSpec(shape, lambda i: (0, 0))


def _blk(shape):
    return pl.BlockSpec(shape, lambda i: (i, 0))


BN = 400


def _node_proj(na, w1b, w1c):
    return pl.pallas_call(
        _node_proj_body,
        grid=(N_NODES // BN,),
        in_specs=[_blk((BN, D_FEAT)), _full((D_FEAT, 16)), _full((D_FEAT, 16))],
        out_specs=[_blk((BN, 16)), _blk((BN, 16))],
        out_shape=[jax.ShapeDtypeStruct((N_NODES, 16), _F32)] * 2,
    )(na, w1b, w1c)


def _pass1(vec, gs, gr, w1a, we2, we3, we4, wenv):
    return pl.pallas_call(
        _pass1_body,
        grid=(N_BLK,),
        in_specs=[_blk((BE, 3)), _blk((BE, 16)), _blk((BE, 16)),
                  _full((N_BASIS + 1, 16)), _full((16, 32)), _full((32, 64)),
                  _full((64, 128)), _full((H, C))],
        out_specs=[_blk((BE, H)), _blk((BE, 9 * C)), _blk((BE, 16))],
        out_shape=[jax.ShapeDtypeStruct((EP, H), _F32),
                   jax.ShapeDtypeStruct((EP, 9 * C), _F32),
                   jax.ShapeDtypeStruct((EP, 16), _F32)],
    )(vec, gs, gr, w1a, we2, we3, we4, wenv)


def _pass2(x1, a0, geom, ww0, wlat1, wlat2, wl1, wl2, wenv1):
    return pl.pallas_call(
        _pass2_body,
        grid=(N_BLK,),
        in_specs=[_blk((BE, H)), _blk((BE, 9 * C)), _blk((BE, 16)),
                  _full((H, 3 * C)), _full((H + 3 * C, H)), _full((H, H)),
                  _full((2 * C, C)), _full((3 * C, C)), _full((H, C))],
        out_specs=[_blk((BE, H)), _blk((BE, 9 * C)), _blk((BE, 8 * C))],
        out_shape=[jax.ShapeDtypeStruct((EP, H), _F32),
                   jax.ShapeDtypeStruct((EP, 9 * C), _F32),
                   jax.ShapeDtypeStruct((EP, 8 * C), _F32)],
    )(x1, a0, geom, ww0, wlat1, wlat2, wl1, wl2, wenv1)


def _pass3(x2, a1, vvt, geom, wlat1, wlat2, wf, wout_row):
    return pl.pallas_call(
        _pass3_body,
        grid=(N_BLK,),
        in_specs=[_blk((BE, H)), _blk((BE, 9 * C)), _blk((BE, 8 * C)),
                  _blk((BE, 16)), _full((H + 3 * C, H)), _full((H, H)),
                  _full((H, H)), _full((1, H))],
        out_specs=_blk((BE, 1)),
        out_shape=jax.ShapeDtypeStruct((EP, 1), _F32),
    )(x2, a1, vvt, geom, wlat1, wlat2, wf, wout_row)


NW = 32
CH = 128
EPW = EP // NW
EPT = EP // 16
HF = 9 * C // 2
NPT = N_NODES // 16


@functools.cache
def _sc_kernels():
    mesh = plsc.VectorSubcoreMesh(core_axis_name="c", subcore_axis_name="s",
                                  num_cores=2, num_subcores=16)
    params = pltpu.CompilerParams(use_tc_tiling_on_sc=False)

    @functools.partial(
        pl.kernel,
        out_type=[jax.ShapeDtypeStruct((EP, 16), _F32),
                  jax.ShapeDtypeStruct((EP, 16), _F32)],
        mesh=mesh,
        compiler_params=params,
        scratch_types=[
            pltpu.VMEM((CH,), jnp.int32),
            pltpu.VMEM((CH,), jnp.int32),
            pltpu.VMEM((CH, 16), _F32),
            pltpu.VMEM((CH, 16), _F32),
            pltpu.SemaphoreType.DMA,
            pltpu.SemaphoreType.DMA,
        ],
    )
    def sc_gather(ps_hbm, pr_hbm, snd_hbm, rcv_hbm, outs_hbm, outr_hbm,
                  idxs_v, idxr_v, rows_s, rows_r, sem_s, sem_r):
        wid = lax.axis_index("s") * 2 + lax.axis_index("c")
        base = wid * EPW

        @pl.loop(0, EPW // CH)
        def _(i):
            e0 = base + i * CH
            pltpu.sync_copy(snd_hbm.at[pl.ds(e0, CH)], idxs_v)
            pltpu.sync_copy(rcv_hbm.at[pl.ds(e0, CH)], idxr_v)
            cs = pltpu.async_copy(ps_hbm.at[idxs_v], rows_s, sem_s)
            cr = pltpu.async_copy(pr_hbm.at[idxr_v], rows_r, sem_r)
            cs.wait()
            cr.wait()
            pltpu.sync_copy(rows_s, outs_hbm.at[pl.ds(e0, CH)])
            pltpu.sync_copy(rows_r, outr_hbm.at[pl.ds(e0, CH)])

    @functools.partial(
        pl.kernel,
        out_type=jax.ShapeDtypeStruct((EP, 9 * C), _F32),
        mesh=mesh,
        compiler_params=params,
        scratch_types=[
            pltpu.VMEM((1, CH), jnp.int32),
            pltpu.VMEM((CH, HF), _F32),
            pltpu.VMEM_SHARED((N_NODES, HF), _F32),
        ],
    )
    def sc_mapback(wy_hbm, snd2d_hbm, zrows_hbm, out_hbm, idx_v, rows_v,
                   acc_sh):
        cid = lax.axis_index("c")
        sid = lax.axis_index("s")
        coff = cid * HF
        pltpu.sync_copy(zrows_hbm.at[pl.ds(sid * NPT, NPT)],
                        acc_sh.at[pl.ds(sid * NPT, NPT)])
        plsc.subcore_barrier()
        base = sid * EPT

        @pl.loop(0, EPT // CH)
        def _(i):
            e0 = base + i * CH
            pltpu.sync_copy(snd2d_hbm.at[pl.ds(e0 // CH, 1)], idx_v)
            pltpu.sync_copy(wy_hbm.at[pl.ds(e0, CH), pl.ds(coff, HF)], rows_v)
            pltpu.sync_copy(rows_v, acc_sh.at[idx_v.at[0]], add=True)

        plsc.subcore_barrier()

        @pl.loop(0, EPT // CH)
        def _(i):
            e0 = base + i * CH
            pltpu.sync_copy(snd2d_hbm.at[pl.ds(e0 // CH, 1)], idx_v)
            pltpu.sync_copy(acc_sh.at[idx_v.at[0]], rows_v)
            pltpu.sync_copy(rows_v, out_hbm.at[pl.ds(e0, CH), pl.ds(coff, HF)])

    return sc_gather, sc_mapback


def kernel(node_attrs, vectors, senders, receivers, W_e1, W_e2, W_e3, W_e4,
           W_w0, W_env_0, W_lat1_0, W_lat2_0, W_l1_0, W_l2_0,
           W_env_1, W_lat1_1, W_lat2_1, W_l1_1, W_l2_1, W_f, W_out):
    senders = senders.astype(jnp.int32)
    receivers = receivers.astype(jnp.int32)
    pad_idx = jnp.arange(PAD, dtype=jnp.int32) * (N_NODES // PAD)
    snd_p = jnp.concatenate([senders, pad_idx])
    rcv_p = jnp.concatenate([receivers, pad_idx])
    vec_p = jnp.concatenate(
        [vectors.astype(_F32), jnp.zeros((PAD, 3), _F32)])

    sc_gather, sc_mapback = _sc_kernels()
    ps, pr = _node_proj(node_attrs.astype(_F32),
                        W_e1[N_BASIS:N_BASIS + D_FEAT],
                        W_e1[N_BASIS + D_FEAT:])
    gs, gr = sc_gather(ps, pr, snd_p, rcv_p)
    w1a = jnp.concatenate([
        W_e1[:N_BASIS] * jnp.asarray(INVSIG_NP.T),
        -jnp.asarray(MU_NP * INVSIG_NP) @ W_e1[:N_BASIS],
    ], axis=0)
    x1, wy0, geom = _pass1(vec_p, gs, gr, w1a, W_e2, W_e3, W_e4, W_env_0)
    snd2d = snd_p.reshape(EP // CH, CH)
    zrows = jnp.zeros((N_NODES, HF), _F32)
    a0 = sc_mapback(wy0, snd2d, zrows)
    x2, wy1, vvt = _pass2(x1, a0, geom, W_w0, W_lat1_0, W_lat2_0, W_l1_0,
                          W_l2_0, W_env_1)
    a1 = sc_mapback(wy1, snd2d, zrows)
    out = _pass3(x2, a1, vvt, geom, W_lat1_1, W_lat2_1, W_f,
                 W_out[:H].reshape(1, H))
    return out[:N_EDGES]


# speedup vs baseline: 8.7764x; 1.0087x over previous
import functools

import numpy as np
import jax
import jax.numpy as jnp
from jax import lax
from jax.experimental import pallas as pl
from jax.experimental.pallas import tpu as pltpu
from jax.experimental.pallas import tpu_sc as plsc

N_NODES = 10000
N_EDGES = 160000
D_FEAT = 128
N_BASIS = 8
C = 32
P = 6
RC = 4.0
ANN = 32.0
H = 128

EP = 163840
BE = 512
N_BLK = EP // BE
PAD = EP - N_EDGES

_trapz = getattr(np, 'trapezoid', None) or np.trapz


def _bessel_np(r, n):
    k = np.arange(1, n + 1)[None, :]
    r = r[:, None]
    safe = np.where(r == 0.0, 1.0, r)
    return np.sqrt(2.0) * np.where(r == 0.0, k * np.pi, np.sin(k * np.pi * safe) / safe)


_r = np.linspace(0.0, 1.0, 1000)
_b = _bessel_np(_r, N_BASIS)
_MU = _trapz(_b, _r, axis=0)
_SIG = _trapz((_b - _MU) ** 2, _r, axis=0) ** 0.5
MU_NP = np.asarray(_MU, dtype=np.float32)[None, :]
INVSIG_NP = np.asarray(1.0 / _SIG, dtype=np.float32)[None, :]

SQRT2 = float(np.sqrt(2.0))
SQRT3 = float(np.sqrt(3.0))
SQRT5 = float(np.sqrt(5.0))
SQRT15 = float(np.sqrt(15.0))
INV_SQRT3 = float(1.0 / np.sqrt(3.0))
INV_SQRT5 = float(1.0 / np.sqrt(5.0))
INV_SQRT_ANN = float(1.0 / np.sqrt(ANN))
INV_SQRT125 = float(1.0 / np.sqrt(1.25))
S2 = float(1.0 / np.sqrt(2.0))
S6 = float(1.0 / np.sqrt(6.0))

_F32 = jnp.float32


def _silu(x):
    return x * jax.nn.sigmoid(x)


def _dot(a, b):
    return jnp.dot(a, b, preferred_element_type=jnp.float32)


def _node_proj_body(na_ref, w1b_ref, w1c_ref, ps_ref, pr_ref):
    na = na_ref[...]
    ps_ref[...] = _dot(na, w1b_ref[...])
    pr_ref[...] = _dot(na, w1c_ref[...])


def _pass1_body(vec_ref, gs_ref, gr_ref, w1a_ref, we2_ref, we3_ref, we4_ref,
                wenv_ref, x1_ref, wy_ref, geom_ref):
    v = vec_ref[...] * (1.0 / RC)
    d2 = jnp.sum(v * v, axis=1, keepdims=True)
    d = jnp.sqrt(d2)
    iszero = d == 0.0
    safe = jnp.where(iszero, 1.0, d)
    kpi = np.float32(np.pi) * (
        lax.broadcasted_iota(jnp.int32, (1, N_BASIS), 1) + 1).astype(
            jnp.float32)
    rb = SQRT2 * jnp.where(iszero, kpi, jnp.sin(kpi * safe) / safe)
    rb9 = jnp.concatenate([rb, jnp.ones_like(d)], axis=1)
    pre1 = _dot(rb9, w1a_ref[...]) + gs_ref[...] + gr_ref[...]
    pre1 = jnp.where(iszero, 0.0, pre1)
    x = _silu(pre1)
    x = _silu(_dot(x, we2_ref[...]))
    x = _silu(_dot(x, we3_ref[...]))
    x = _dot(x, we4_ref[...])
    d6 = d2 * d2 * d2
    d7 = d6 * d
    d8 = d7 * d
    env = jnp.where(d < 1.0, 1.0 - 28.0 * d6 + 48.0 * d7 - 21.0 * d8, 0.0)
    x = env * x
    rh = v / safe
    xx = rh[:, 0:1]
    yy = rh[:, 1:2]
    zz = rh[:, 2:3]
    Y = jnp.concatenate([
        jnp.ones_like(xx), SQRT3 * xx, SQRT3 * yy, SQRT3 * zz,
        SQRT15 * xx * yy, SQRT15 * yy * zz, (SQRT5 / 2.0) * (3.0 * zz * zz - 1.0),
        SQRT15 * xx * zz, (SQRT15 / 2.0) * (xx * xx - yy * yy)
    ], axis=1)
    x1_ref[...] = x
    we = _dot(x, wenv_ref[...])
    wy_ref[...] = jnp.concatenate([we * Y[:, j:j + 1] for j in range(9)], axis=1)
    geom_ref[...] = jnp.concatenate(
        [env, Y, jnp.zeros_like(Y[:, :6])], axis=1)


def _pass2_body(x1_ref, a_ref, geom_ref, wcomb_ref, wlat1_ref, wlat2e_ref,
                wall_ref, x2_ref, wy_ref, vvt_ref):
    x1 = x1_ref[...]
    g = geom_ref[...]
    env = g[:, 0:1]
    wfull = _dot(x1, wcomb_ref[...])
    s = wfull[:, 0:C]
    wv = wfull[:, C:2 * C]
    wt = wfull[:, 2 * C:3 * C]
    x1we = wfull[:, 3 * C:4 * C]
    vvg = [g[:, 2 + k:3 + k] * wv for k in range(3)]
    tg = [g[:, 5 + m:6 + m] * wt for m in range(5)]
    A = a_ref[...] * INV_SQRT_ANN
    a0 = A[:, 0:C]
    a1 = [A[:, C * (1 + k):C * (2 + k)] for k in range(3)]
    a2 = [A[:, C * (4 + m):C * (5 + m)] for m in range(5)]
    p00 = a0 * s
    p11 = (a1[0] * vvg[0] + a1[1] * vvg[1] + a1[2] * vvg[2]) * INV_SQRT3
    p22 = (a2[0] * tg[0] + a2[1] * tg[1] + a2[2] * tg[2] + a2[3] * tg[3]
           + a2[4] * tg[4]) * INV_SQRT5
    xl = jnp.concatenate([x1, p00, p11, p22], axis=1)
    h = _silu(_dot(xl, wlat1_ref[...]))
    yz = _dot(h, wlat2e_ref[...])
    x2 = (x1 + (0.5 * env) * yz[:, 0:H]) * INV_SQRT125
    x2_ref[...] = x2
    we = (x1we + (0.5 * env) * yz[:, H:H + C]) * INV_SQRT125
    ax, ay, az = a1
    bx, by, bz = vvg
    r11 = [S2 * (ax * by + ay * bx),
           S2 * (ay * bz + az * by),
           S6 * (2.0 * az * bz - ax * bx - ay * by),
           S2 * (ax * bz + az * bx),
           S2 * (ax * bx - ay * by)]
    paths = [a0 * vvg[0], ax * s, a0 * vvg[1], ay * s, a0 * vvg[2], az * s]
    for m in range(5):
        paths += [a0 * tg[m], a2[m] * s, r11[m]]
    vvt_ref[...] = _dot(jnp.concatenate(paths, axis=1), wall_ref[...])
    wy_ref[...] = jnp.concatenate([we * g[:, 1 + j:2 + j] for j in range(9)],
                                  axis=1)


def _pass3_body(x2_ref, a_ref, vvt_ref, geom_ref, wlat1r_ref, fo_ref,
                go_ref, o_ref):
    x2 = x2_ref[...]
    env = geom_ref[:, 0:1]
    A = a_ref[...] * INV_SQRT_ANN
    a1 = [A[:, C * (1 + k):C * (2 + k)] for k in range(3)]
    a2 = [A[:, C * (4 + m):C * (5 + m)] for m in range(5)]
    vvt = vvt_ref[...]
    vvg = [vvt[:, C * k:C * (k + 1)] for k in range(3)]
    tg = [vvt[:, 3 * C + C * m:3 * C + C * (m + 1)] for m in range(5)]
    p11 = (a1[0] * vvg[0] + a1[1] * vvg[1] + a1[2] * vvg[2]) * INV_SQRT3
    p22 = (a2[0] * tg[0] + a2[1] * tg[1] + a2[2] * tg[2] + a2[3] * tg[3]
           + a2[4] * tg[4]) * INV_SQRT5
    xl = jnp.concatenate([x2, p11, p22], axis=1)
    h = _silu(_dot(xl, wlat1r_ref[...]))
    o = (jnp.sum(x2 * fo_ref[...], axis=1, keepdims=True)
         + (0.5 * env) * jnp.sum(h * go_ref[...], axis=1, keepdims=True))
    o_ref[...] = o * INV_SQRT125


def _full(shape):
    return pl.BlockSpec(shape, lambda i: (0, 0))


def _blk(shape):
    return pl.BlockSpec(shape, lambda i: (i, 0))


BN = 400


def _node_proj(na, w1b, w1c):
    return pl.pallas_call(
        _node_proj_body,
        grid=(N_NODES // BN,),
        in_specs=[_blk((BN, D_FEAT)), _full((D_FEAT, 16)), _full((D_FEAT, 16))],
        out_specs=[_blk((BN, 16)), _blk((BN, 16))],
        out_shape=[jax.ShapeDtypeStruct((N_NODES, 16), _F32)] * 2,
    )(na, w1b, w1c)


def _pass1(vec, gs, gr, w1a, we2, we3, we4, wenv):
    return pl.pallas_call(
        _pass1_body,
        grid=(N_BLK,),
        in_specs=[_blk((BE, 3)), _blk((BE, 16)), _blk((BE, 16)),
                  _full((N_BASIS + 1, 16)), _full((16, 32)), _full((32, 64)),
                  _full((64, 128)), _full((H, C))],
        out_specs=[_blk((BE, H)), _blk((BE, 9 * C)), _blk((BE, 16))],
        out_shape=[jax.ShapeDtypeStruct((EP, H), _F32),
                   jax.ShapeDtypeStruct((EP, 9 * C), _F32),
                   jax.ShapeDtypeStruct((EP, 16), _F32)],
    )(vec, gs, gr, w1a, we2, we3, we4, wenv)


def _pass2(x1, a0, geom, wcomb, wlat1, wlat2e, wall):
    return pl.pallas_call(
        _pass2_body,
        grid=(N_BLK,),
        in_specs=[_blk((BE, H)), _blk((BE, 9 * C)), _blk((BE, 16)),
                  _full((H, H)), _full((H + 3 * C, H)), _full((H, H + C)),
                  _full((21 * C, 8 * C))],
        out_specs=[_blk((BE, H)), _blk((BE, 9 * C)), _blk((BE, 8 * C))],
        out_shape=[jax.ShapeDtypeStruct((EP, H), _F32),
                   jax.ShapeDtypeStruct((EP, 9 * C), _F32),
                   jax.ShapeDtypeStruct((EP, 8 * C), _F32)],
    )(x1, a0, geom, wcomb, wlat1, wlat2e, wall)


def _pass3(x2, a1, vvt, geom, wlat1r, fo, go):
    return pl.pallas_call(
        _pass3_body,
        grid=(N_BLK,),
        in_specs=[_blk((BE, H)), _blk((BE, 9 * C)), _blk((BE, 8 * C)),
                  _blk((BE, 16)), _full((H + 2 * C, H)), _full((1, H)),
                  _full((1, H))],
        out_specs=_blk((BE, 1)),
        out_shape=jax.ShapeDtypeStruct((EP, 1), _F32),
    )(x2, a1, vvt, geom, wlat1r, fo, go)


NW = 32
CH = 128
EPW = EP // NW
EPT = EP // 16
HF = 9 * C // 2
NPT = N_NODES // 16


@functools.cache
def _sc_kernels():
    mesh = plsc.VectorSubcoreMesh(core_axis_name="c", subcore_axis_name="s",
                                  num_cores=2, num_subcores=16)
    params = pltpu.CompilerParams(use_tc_tiling_on_sc=False)

    @functools.partial(
        pl.kernel,
        out_type=[jax.ShapeDtypeStruct((EP, 16), _F32),
                  jax.ShapeDtypeStruct((EP, 16), _F32)],
        mesh=mesh,
        compiler_params=params,
        scratch_types=[
            pltpu.VMEM((CH,), jnp.int32),
            pltpu.VMEM((CH,), jnp.int32),
            pltpu.VMEM((CH, 16), _F32),
            pltpu.VMEM((CH, 16), _F32),
            pltpu.SemaphoreType.DMA,
            pltpu.SemaphoreType.DMA,
        ],
    )
    def sc_gather(ps_hbm, pr_hbm, snd_hbm, rcv_hbm, outs_hbm, outr_hbm,
                  idxs_v, idxr_v, rows_s, rows_r, sem_s, sem_r):
        wid = lax.axis_index("s") * 2 + lax.axis_index("c")
        base = wid * EPW

        @pl.loop(0, EPW // CH)
        def _(i):
            e0 = base + i * CH
            pltpu.sync_copy(snd_hbm.at[pl.ds(e0, CH)], idxs_v)
            pltpu.sync_copy(rcv_hbm.at[pl.ds(e0, CH)], idxr_v)
            cs = pltpu.async_copy(ps_hbm.at[idxs_v], rows_s, sem_s)
            cr = pltpu.async_copy(pr_hbm.at[idxr_v], rows_r, sem_r)
            cs.wait()
            cr.wait()
            pltpu.sync_copy(rows_s, outs_hbm.at[pl.ds(e0, CH)])
            pltpu.sync_copy(rows_r, outr_hbm.at[pl.ds(e0, CH)])

    @functools.partial(
        pl.kernel,
        out_type=jax.ShapeDtypeStruct((EP, 9 * C), _F32),
        mesh=mesh,
        compiler_params=params,
        scratch_types=[
            pltpu.VMEM((1, CH), jnp.int32),
            pltpu.VMEM((CH, HF), _F32),
            pltpu.VMEM_SHARED((N_NODES, HF), _F32),
        ],
    )
    def sc_mapback(wy_hbm, snd2d_hbm, zrows_hbm, out_hbm, idx_v, rows_v,
                   acc_sh):
        cid = lax.axis_index("c")
        sid = lax.axis_index("s")
        coff = cid * HF
        pltpu.sync_copy(zrows_hbm.at[pl.ds(sid * NPT, NPT)],
                        acc_sh.at[pl.ds(sid * NPT, NPT)])
        plsc.subcore_barrier()
        base = sid * EPT

        @pl.loop(0, EPT // CH)
        def _(i):
            e0 = base + i * CH
            pltpu.sync_copy(snd2d_hbm.at[pl.ds(e0 // CH, 1)], idx_v)
            pltpu.sync_copy(wy_hbm.at[pl.ds(e0, CH), pl.ds(coff, HF)], rows_v)
            pltpu.sync_copy(rows_v, acc_sh.at[idx_v.at[0]], add=True)

        plsc.subcore_barrier()

        @pl.loop(0, EPT // CH)
        def _(i):
            e0 = base + i * CH
            pltpu.sync_copy(snd2d_hbm.at[pl.ds(e0 // CH, 1)], idx_v)
            pltpu.sync_copy(acc_sh.at[idx_v.at[0]], rows_v)
            pltpu.sync_copy(rows_v, out_hbm.at[pl.ds(e0, CH), pl.ds(coff, HF)])

    return sc_gather, sc_mapback


def kernel(node_attrs, vectors, senders, receivers, W_e1, W_e2, W_e3, W_e4,
           W_w0, W_env_0, W_lat1_0, W_lat2_0, W_l1_0, W_l2_0,
           W_env_1, W_lat1_1, W_lat2_1, W_l1_1, W_l2_1, W_f, W_out):
    senders = senders.astype(jnp.int32)
    receivers = receivers.astype(jnp.int32)
    pad_idx = jnp.arange(PAD, dtype=jnp.int32) * (N_NODES // PAD)
    snd_p = jnp.concatenate([senders, pad_idx])
    rcv_p = jnp.concatenate([receivers, pad_idx])
    vec_p = jnp.concatenate(
        [vectors.astype(_F32), jnp.zeros((PAD, 3), _F32)])

    sc_gather, sc_mapback = _sc_kernels()
    ps, pr = _node_proj(node_attrs.astype(_F32),
                        W_e1[N_BASIS:N_BASIS + D_FEAT],
                        W_e1[N_BASIS + D_FEAT:])
    gs, gr = sc_gather(ps, pr, snd_p, rcv_p)
    w1a = jnp.concatenate([
        W_e1[:N_BASIS] * jnp.asarray(INVSIG_NP.T),
        -jnp.asarray(MU_NP * INVSIG_NP) @ W_e1[:N_BASIS],
    ], axis=0)
    x1, wy0, geom = _pass1(vec_p, gs, gr, w1a, W_e2, W_e3, W_e4, W_env_0)
    snd2d = snd_p.reshape(EP // CH, CH)
    zrows = jnp.zeros((N_NODES, HF), _F32)
    a0 = sc_mapback(wy0, snd2d, zrows)
    wcomb = jnp.concatenate([W_w0, W_env_1], axis=1)
    wlat2e = jnp.concatenate([W_lat2_0, W_lat2_0 @ W_env_1], axis=1)
    wall = jnp.zeros((21 * C, 8 * C), _F32)
    for k in range(3):
        wall = wall.at[2 * C * k:2 * C * (k + 1),
                       C * k:C * (k + 1)].set(W_l1_0)
    for m in range(5):
        wall = wall.at[6 * C + 3 * C * m:6 * C + 3 * C * (m + 1),
                       3 * C + C * m:3 * C + C * (m + 1)].set(W_l2_0)
    x2, wy1, vvt = _pass2(x1, a0, geom, wcomb, W_lat1_0, wlat2e, wall)
    a1 = sc_mapback(wy1, snd2d, zrows)
    fo = W_f @ W_out[:H]
    go = W_lat2_1 @ fo
    wlat1r = jnp.concatenate([W_lat1_1[:H], W_lat1_1[H + C:]], axis=0)
    out = _pass3(x2, a1, vvt, geom, wlat1r, fo.reshape(1, H), go.reshape(1, H))
    return out[:N_EDGES]


# speedup vs baseline: 10.2399x; 1.1668x over previous
import functools

import numpy as np
import jax
import jax.numpy as jnp
from jax import lax
from jax.experimental import pallas as pl
from jax.experimental.pallas import tpu as pltpu
from jax.experimental.pallas import tpu_sc as plsc

N_NODES = 10000
N_EDGES = 160000
D_FEAT = 128
N_BASIS = 8
C = 32
P = 6
RC = 4.0
ANN = 32.0
H = 128

EP = 163840
BE = 512
N_BLK = EP // BE
PAD = EP - N_EDGES

_trapz = getattr(np, 'trapezoid', None) or np.trapz


def _bessel_np(r, n):
    k = np.arange(1, n + 1)[None, :]
    r = r[:, None]
    safe = np.where(r == 0.0, 1.0, r)
    return np.sqrt(2.0) * np.where(r == 0.0, k * np.pi, np.sin(k * np.pi * safe) / safe)


_r = np.linspace(0.0, 1.0, 1000)
_b = _bessel_np(_r, N_BASIS)
_MU = _trapz(_b, _r, axis=0)
_SIG = _trapz((_b - _MU) ** 2, _r, axis=0) ** 0.5
MU_NP = np.asarray(_MU, dtype=np.float32)[None, :]
INVSIG_NP = np.asarray(1.0 / _SIG, dtype=np.float32)[None, :]

SQRT2 = float(np.sqrt(2.0))
SQRT3 = float(np.sqrt(3.0))
SQRT5 = float(np.sqrt(5.0))
SQRT15 = float(np.sqrt(15.0))
INV_SQRT3 = float(1.0 / np.sqrt(3.0))
INV_SQRT5 = float(1.0 / np.sqrt(5.0))
INV_SQRT_ANN = float(1.0 / np.sqrt(ANN))
INV_SQRT125 = float(1.0 / np.sqrt(1.25))
S2 = float(1.0 / np.sqrt(2.0))
S6 = float(1.0 / np.sqrt(6.0))

_F32 = jnp.float32

R_NP = np.zeros((9, 9 * C), dtype=np.float32)
for _j in range(9):
    R_NP[_j, C * _j:C * (_j + 1)] = 1.0
TMAT_NP = np.zeros((2 * C, 16 * C), dtype=np.float32)
for _k in range(3):
    TMAT_NP[np.arange(C), C * _k + np.arange(C)] = 1.0
    TMAT_NP[C + np.arange(C), 8 * C + C * _k + np.arange(C)] = 1.0
for _m in range(5):
    TMAT_NP[np.arange(C), 3 * C + C * _m + np.arange(C)] = 1.0
    TMAT_NP[C + np.arange(C), 11 * C + C * _m + np.arange(C)] = 1.0


def _silu(x):
    return x * jax.nn.sigmoid(x)


def _dot(a, b):
    return jnp.dot(a, b, preferred_element_type=jnp.float32)


def _node_proj_body(na_ref, w1b_ref, w1c_ref, ps_ref, pr_ref):
    na = na_ref[...]
    ps_ref[...] = _dot(na, w1b_ref[...])
    pr_ref[...] = _dot(na, w1c_ref[...])


def _pass1_body(vec_ref, gs_ref, gr_ref, w1a_ref, we2_ref, we3_ref, we4_ref,
                wenv9_ref, rmat_ref, x1_ref, wy_ref, geom_ref):
    v = vec_ref[...] * (1.0 / RC)
    d2 = jnp.sum(v * v, axis=1, keepdims=True)
    d = jnp.sqrt(d2)
    iszero = d == 0.0
    safe = jnp.where(iszero, 1.0, d)
    kpi = np.float32(np.pi) * (
        lax.broadcasted_iota(jnp.int32, (1, N_BASIS), 1) + 1).astype(
            jnp.float32)
    rb = SQRT2 * jnp.where(iszero, kpi, jnp.sin(kpi * safe) / safe)
    rb9 = jnp.concatenate([rb, jnp.ones_like(d)], axis=1)
    pre1 = _dot(rb9, w1a_ref[...]) + gs_ref[...] + gr_ref[...]
    pre1 = jnp.where(iszero, 0.0, pre1)
    x = _silu(pre1)
    x = _silu(_dot(x, we2_ref[...]))
    x = _silu(_dot(x, we3_ref[...]))
    x = _dot(x, we4_ref[...])
    d6 = d2 * d2 * d2
    d7 = d6 * d
    d8 = d7 * d
    env = jnp.where(d < 1.0, 1.0 - 28.0 * d6 + 48.0 * d7 - 21.0 * d8, 0.0)
    x = env * x
    rh = v / safe
    xx = rh[:, 0:1]
    yy = rh[:, 1:2]
    zz = rh[:, 2:3]
    Y = jnp.concatenate([
        jnp.ones_like(xx), SQRT3 * xx, SQRT3 * yy, SQRT3 * zz,
        SQRT15 * xx * yy, SQRT15 * yy * zz, (SQRT5 / 2.0) * (3.0 * zz * zz - 1.0),
        SQRT15 * xx * zz, (SQRT15 / 2.0) * (xx * xx - yy * yy)
    ], axis=1)
    x1_ref[...] = x
    ye = _dot(Y, rmat_ref[...])
    wy_ref[...] = ye * _dot(x, wenv9_ref[...])
    geom_ref[...] = jnp.concatenate(
        [env, Y, jnp.zeros_like(Y[:, :6])], axis=1)


def _pass2_body(x1_ref, a_ref, geom_ref, wcomb_ref, wlat1_ref, wlat2e_ref,
                wall_ref, rmat_ref, tmat_ref, x2_ref, wy_ref, vvt_ref):
    x1 = x1_ref[...]
    g = geom_ref[...]
    env = g[:, 0:1]
    ye = _dot(g[:, 1:10], rmat_ref[...])
    wfull = _dot(x1, wcomb_ref[...])
    s = wfull[:, 0:C]
    wv3 = wfull[:, C:4 * C]
    wt5 = wfull[:, 4 * C:9 * C]
    x1we = wfull[:, 9 * C:18 * C]
    vvgw = ye[:, C:4 * C] * wv3
    tgw = ye[:, 4 * C:9 * C] * wt5
    A = a_ref[...] * INV_SQRT_ANN
    a0 = A[:, 0:C]
    a1w = A[:, C:4 * C]
    a2w = A[:, 4 * C:9 * C]
    pv = a1w * vvgw
    pt = a2w * tgw
    p00 = a0 * s
    p11 = (pv[:, 0:C] + pv[:, C:2 * C] + pv[:, 2 * C:3 * C]) * INV_SQRT3
    p22 = (pt[:, 0:C] + pt[:, C:2 * C] + pt[:, 2 * C:3 * C]
           + pt[:, 3 * C:4 * C] + pt[:, 4 * C:5 * C]) * INV_SQRT5
    xl = jnp.concatenate([x1, p00, p11, p22], axis=1)
    h = _silu(_dot(xl, wlat1_ref[...]))
    yz = _dot(h, wlat2e_ref[...])
    x2 = (x1 + (0.5 * env) * yz[:, 0:H]) * INV_SQRT125
    x2_ref[...] = x2
    wew = (x1we + (0.5 * env) * yz[:, H:H + 9 * C]) * INV_SQRT125
    wy_ref[...] = ye * wew
    ts = _dot(jnp.concatenate([a0, s], axis=1), tmat_ref[...])
    a0_3 = ts[:, 0:3 * C]
    a0_5 = ts[:, 3 * C:8 * C]
    s3 = ts[:, 8 * C:11 * C]
    s5 = ts[:, 11 * C:16 * C]
    ax, ay, az = a1w[:, 0:C], a1w[:, C:2 * C], a1w[:, 2 * C:3 * C]
    bx, by, bz = vvgw[:, 0:C], vvgw[:, C:2 * C], vvgw[:, 2 * C:3 * C]
    r11w = jnp.concatenate(
        [S2 * (ax * by + ay * bx),
         S2 * (ay * bz + az * by),
         S6 * (2.0 * az * bz - ax * bx - ay * by),
         S2 * (ax * bz + az * bx),
         S2 * (ax * bx - ay * by)], axis=1)
    xall = jnp.concatenate(
        [a0_3 * vvgw, a1w * s3, a0_5 * tgw, a2w * s5, r11w], axis=1)
    vvt_ref[...] = _dot(xall, wall_ref[...])


def _pass3_body(x2_ref, a_ref, vvt_ref, geom_ref, wlat1r_ref, fo_ref,
                go_ref, o_ref):
    x2 = x2_ref[...]
    env = geom_ref[:, 0:1]
    A = a_ref[...] * INV_SQRT_ANN
    vvt = vvt_ref[...]
    pv = A[:, C:4 * C] * vvt[:, 0:3 * C]
    pt = A[:, 4 * C:9 * C] * vvt[:, 3 * C:8 * C]
    p11 = (pv[:, 0:C] + pv[:, C:2 * C] + pv[:, 2 * C:3 * C]) * INV_SQRT3
    p22 = (pt[:, 0:C] + pt[:, C:2 * C] + pt[:, 2 * C:3 * C]
           + pt[:, 3 * C:4 * C] + pt[:, 4 * C:5 * C]) * INV_SQRT5
    xl = jnp.concatenate([x2, p11, p22], axis=1)
    h = _silu(_dot(xl, wlat1r_ref[...]))
    o = (jnp.sum(x2 * fo_ref[...], axis=1, keepdims=True)
         + (0.5 * env) * jnp.sum(h * go_ref[...], axis=1, keepdims=True))
    o_ref[...] = o * INV_SQRT125


def _full(shape):
    return pl.BlockSpec(shape, lambda i: (0, 0))


def _blk(shape):
    return pl.BlockSpec(shape, lambda i: (i, 0))


BN = 400


def _node_proj(na, w1b, w1c):
    return pl.pallas_call(
        _node_proj_body,
        grid=(N_NODES // BN,),
        in_specs=[_blk((BN, D_FEAT)), _full((D_FEAT, 16)), _full((D_FEAT, 16))],
        out_specs=[_blk((BN, 16)), _blk((BN, 16))],
        out_shape=[jax.ShapeDtypeStruct((N_NODES, 16), _F32)] * 2,
    )(na, w1b, w1c)


def _pass1(vec, gs, gr, w1a, we2, we3, we4, wenv9, rmat):
    return pl.pallas_call(
        _pass1_body,
        grid=(N_BLK,),
        in_specs=[_blk((BE, 3)), _blk((BE, 16)), _blk((BE, 16)),
                  _full((N_BASIS + 1, 16)), _full((16, 32)), _full((32, 64)),
                  _full((64, 128)), _full((H, 9 * C)), _full((9, 9 * C))],
        out_specs=[_blk((BE, H)), _blk((BE, 9 * C)), _blk((BE, 16))],
        out_shape=[jax.ShapeDtypeStruct((EP, H), _F32),
                   jax.ShapeDtypeStruct((EP, 9 * C), _F32),
                   jax.ShapeDtypeStruct((EP, 16), _F32)],
    )(vec, gs, gr, w1a, we2, we3, we4, wenv9, rmat)


def _pass2(x1, a0, geom, wcomb, wlat1, wlat2e, wall, rmat, tmat):
    return pl.pallas_call(
        _pass2_body,
        grid=(N_BLK,),
        in_specs=[_blk((BE, H)), _blk((BE, 9 * C)), _blk((BE, 16)),
                  _full((H, 18 * C)), _full((H + 3 * C, H)),
                  _full((H, H + 9 * C)), _full((21 * C, 8 * C)),
                  _full((9, 9 * C)), _full((2 * C, 16 * C))],
        out_specs=[_blk((BE, H)), _blk((BE, 9 * C)), _blk((BE, 8 * C))],
        out_shape=[jax.ShapeDtypeStruct((EP, H), _F32),
                   jax.ShapeDtypeStruct((EP, 9 * C), _F32),
                   jax.ShapeDtypeStruct((EP, 8 * C), _F32)],
    )(x1, a0, geom, wcomb, wlat1, wlat2e, wall, rmat, tmat)


def _pass3(x2, a1, vvt, geom, wlat1r, fo, go):
    return pl.pallas_call(
        _pass3_body,
        grid=(N_BLK,),
        in_specs=[_blk((BE, H)), _blk((BE, 9 * C)), _blk((BE, 8 * C)),
                  _blk((BE, 16)), _full((H + 2 * C, H)), _full((1, H)),
                  _full((1, H))],
        out_specs=_blk((BE, 1)),
        out_shape=jax.ShapeDtypeStruct((EP, 1), _F32),
    )(x2, a1, vvt, geom, wlat1r, fo, go)


NW = 32
CH = 128
EPW = EP // NW
EPT = EP // 16
HF = 9 * C // 2
NPT = N_NODES // 16


@functools.cache
def _sc_kernels():
    mesh = plsc.VectorSubcoreMesh(core_axis_name="c", subcore_axis_name="s",
                                  num_cores=2, num_subcores=16)
    params = pltpu.CompilerParams(use_tc_tiling_on_sc=False)

    @functools.partial(
        pl.kernel,
        out_type=[jax.ShapeDtypeStruct((EP, 16), _F32),
                  jax.ShapeDtypeStruct((EP, 16), _F32)],
        mesh=mesh,
        compiler_params=params,
        scratch_types=[
            pltpu.VMEM((CH,), jnp.int32),
            pltpu.VMEM((CH,), jnp.int32),
            pltpu.VMEM((CH, 16), _F32),
            pltpu.VMEM((CH, 16), _F32),
            pltpu.SemaphoreType.DMA,
            pltpu.SemaphoreType.DMA,
        ],
    )
    def sc_gather(ps_hbm, pr_hbm, snd_hbm, rcv_hbm, outs_hbm, outr_hbm,
                  idxs_v, idxr_v, rows_s, rows_r, sem_s, sem_r):
        wid = lax.axis_index("s") * 2 + lax.axis_index("c")
        base = wid * EPW

        @pl.loop(0, EPW // CH)
        def _(i):
            e0 = base + i * CH
            pltpu.sync_copy(snd_hbm.at[pl.ds(e0, CH)], idxs_v)
            pltpu.sync_copy(rcv_hbm.at[pl.ds(e0, CH)], idxr_v)
            cs = pltpu.async_copy(ps_hbm.at[idxs_v], rows_s, sem_s)
            cr = pltpu.async_copy(pr_hbm.at[idxr_v], rows_r, sem_r)
            cs.wait()
            cr.wait()
            pltpu.sync_copy(rows_s, outs_hbm.at[pl.ds(e0, CH)])
            pltpu.sync_copy(rows_r, outr_hbm.at[pl.ds(e0, CH)])

    @functools.partial(
        pl.kernel,
        out_type=jax.ShapeDtypeStruct((EP, 9 * C), _F32),
        mesh=mesh,
        compiler_params=params,
        scratch_types=[
            pltpu.VMEM((1, CH), jnp.int32),
            pltpu.VMEM((CH, HF), _F32),
            pltpu.VMEM_SHARED((N_NODES, HF), _F32),
        ],
    )
    def sc_mapback(wy_hbm, snd2d_hbm, zrows_hbm, out_hbm, idx_v, rows_v,
                   acc_sh):
        cid = lax.axis_index("c")
        sid = lax.axis_index("s")
        coff = cid * HF
        pltpu.sync_copy(zrows_hbm.at[pl.ds(sid * NPT, NPT)],
                        acc_sh.at[pl.ds(sid * NPT, NPT)])
        plsc.subcore_barrier()
        base = sid * EPT

        @pl.loop(0, EPT // CH)
        def _(i):
            e0 = base + i * CH
            pltpu.sync_copy(snd2d_hbm.at[pl.ds(e0 // CH, 1)], idx_v)
            pltpu.sync_copy(wy_hbm.at[pl.ds(e0, CH), pl.ds(coff, HF)], rows_v)
            pltpu.sync_copy(rows_v, acc_sh.at[idx_v.at[0]], add=True)

        plsc.subcore_barrier()

        @pl.loop(0, EPT // CH)
        def _(i):
            e0 = base + i * CH
            pltpu.sync_copy(snd2d_hbm.at[pl.ds(e0 // CH, 1)], idx_v)
            pltpu.sync_copy(acc_sh.at[idx_v.at[0]], rows_v)
            pltpu.sync_copy(rows_v, out_hbm.at[pl.ds(e0, CH), pl.ds(coff, HF)])

    return sc_gather, sc_mapback


def kernel(node_attrs, vectors, senders, receivers, W_e1, W_e2, W_e3, W_e4,
           W_w0, W_env_0, W_lat1_0, W_lat2_0, W_l1_0, W_l2_0,
           W_env_1, W_lat1_1, W_lat2_1, W_l1_1, W_l2_1, W_f, W_out):
    senders = senders.astype(jnp.int32)
    receivers = receivers.astype(jnp.int32)
    pad_idx = jnp.arange(PAD, dtype=jnp.int32) * (N_NODES // PAD)
    snd_p = jnp.concatenate([senders, pad_idx])
    rcv_p = jnp.concatenate([receivers, pad_idx])
    vec_p = jnp.concatenate(
        [vectors.astype(_F32), jnp.zeros((PAD, 3), _F32)])

    sc_gather, sc_mapback = _sc_kernels()
    ps, pr = _node_proj(node_attrs.astype(_F32),
                        W_e1[N_BASIS:N_BASIS + D_FEAT],
                        W_e1[N_BASIS + D_FEAT:])
    gs, gr = sc_gather(ps, pr, snd_p, rcv_p)
    w1a = jnp.concatenate([
        W_e1[:N_BASIS] * jnp.asarray(INVSIG_NP.T),
        -jnp.asarray(MU_NP * INVSIG_NP) @ W_e1[:N_BASIS],
    ], axis=0)
    rmat = jnp.asarray(R_NP)
    tmat = jnp.asarray(TMAT_NP)
    wenv9 = jnp.tile(W_env_0, (1, 9))
    x1, wy0, geom = _pass1(vec_p, gs, gr, w1a, W_e2, W_e3, W_e4, wenv9, rmat)
    snd2d = snd_p.reshape(EP // CH, CH)
    zrows = jnp.zeros((N_NODES, HF), _F32)
    a0 = sc_mapback(wy0, snd2d, zrows)
    wcomb = jnp.concatenate([
        W_w0[:, 0:C], jnp.tile(W_w0[:, C:2 * C], (1, 3)),
        jnp.tile(W_w0[:, 2 * C:3 * C], (1, 5)), jnp.tile(W_env_1, (1, 9)),
    ], axis=1)
    wlat2e = jnp.concatenate(
        [W_lat2_0, jnp.tile(W_lat2_0 @ W_env_1, (1, 9))], axis=1)
    wall = jnp.zeros((21 * C, 8 * C), _F32)
    for k in range(3):
        wall = wall.at[C * k:C * (k + 1), C * k:C * (k + 1)].set(
            W_l1_0[:C])
        wall = wall.at[3 * C + C * k:3 * C + C * (k + 1),
                       C * k:C * (k + 1)].set(W_l1_0[C:])
    for m in range(5):
        wall = wall.at[6 * C + C * m:6 * C + C * (m + 1),
                       3 * C + C * m:3 * C + C * (m + 1)].set(
            W_l2_0[:C])
        wall = wall.at[11 * C + C * m:11 * C + C * (m + 1),
                       3 * C + C * m:3 * C + C * (m + 1)].set(
            W_l2_0[C:2 * C])
        wall = wall.at[16 * C + C * m:16 * C + C * (m + 1),
                       3 * C + C * m:3 * C + C * (m + 1)].set(
            W_l2_0[2 * C:])
    x2, wy1, vvt = _pass2(x1, a0, geom, wcomb, W_lat1_0, wlat2e, wall,
                          rmat, tmat)
    a1 = sc_mapback(wy1, snd2d, zrows)
    fo = W_f @ W_out[:H]
    go = W_lat2_1 @ fo
    wlat1r = jnp.concatenate([W_lat1_1[:H], W_lat1_1[H + C:]], axis=0)
    out = _pass3(x2, a1, vvt, geom, wlat1r, fo.reshape(1, H), go.reshape(1, H))
    return out[:N_EDGES]


# speedup vs baseline: 11.2930x; 1.1029x over previous
import functools

import numpy as np
import jax
import jax.numpy as jnp
from jax import lax
from jax.experimental import pallas as pl
from jax.experimental.pallas import tpu as pltpu
from jax.experimental.pallas import tpu_sc as plsc

N_NODES = 10000
N_EDGES = 160000
D_FEAT = 128
N_BASIS = 8
C = 32
P = 6
RC = 4.0
ANN = 32.0
H = 128

EP = 163840
BE = 512
N_BLK = EP // BE
PAD = EP - N_EDGES

_trapz = getattr(np, 'trapezoid', None) or np.trapz


def _bessel_np(r, n):
    k = np.arange(1, n + 1)[None, :]
    r = r[:, None]
    safe = np.where(r == 0.0, 1.0, r)
    return np.sqrt(2.0) * np.where(r == 0.0, k * np.pi, np.sin(k * np.pi * safe) / safe)


_r = np.linspace(0.0, 1.0, 1000)
_b = _bessel_np(_r, N_BASIS)
_MU = _trapz(_b, _r, axis=0)
_SIG = _trapz((_b - _MU) ** 2, _r, axis=0) ** 0.5
MU_NP = np.asarray(_MU, dtype=np.float32)[None, :]
INVSIG_NP = np.asarray(1.0 / _SIG, dtype=np.float32)[None, :]

SQRT2 = float(np.sqrt(2.0))
SQRT3 = float(np.sqrt(3.0))
SQRT5 = float(np.sqrt(5.0))
SQRT15 = float(np.sqrt(15.0))
INV_SQRT3 = float(1.0 / np.sqrt(3.0))
INV_SQRT5 = float(1.0 / np.sqrt(5.0))
INV_SQRT_ANN = float(1.0 / np.sqrt(ANN))
INV_SQRT125 = float(1.0 / np.sqrt(1.25))
S2 = float(1.0 / np.sqrt(2.0))
S6 = float(1.0 / np.sqrt(6.0))

_F32 = jnp.float32

R_NP = np.zeros((9, 9 * C), dtype=np.float32)
for _j in range(9):
    R_NP[_j, C * _j:C * (_j + 1)] = 1.0
TMAT_NP = np.zeros((2 * C, 16 * C), dtype=np.float32)
for _k in range(3):
    TMAT_NP[np.arange(C), C * _k + np.arange(C)] = 1.0
    TMAT_NP[C + np.arange(C), 8 * C + C * _k + np.arange(C)] = 1.0
for _m in range(5):
    TMAT_NP[np.arange(C), 3 * C + C * _m + np.arange(C)] = 1.0
    TMAT_NP[C + np.arange(C), 11 * C + C * _m + np.arange(C)] = 1.0


def _silu(x):
    return x * jax.nn.sigmoid(x)


def _dot(a, b):
    return jnp.dot(a, b, preferred_element_type=jnp.float32)


def _node_proj_body(na_ref, w1b_ref, w1c_ref, ps_ref, pr_ref):
    na = na_ref[...]
    ps_ref[...] = _dot(na, w1b_ref[...])
    pr_ref[...] = _dot(na, w1c_ref[...])


def _pass1_body(vec_ref, gs_ref, gr_ref, w1a_ref, we2_ref, we3_ref, we4_ref,
                wenv9_ref, rmat_ref, x1_ref, wy_ref, geom_ref):
    v = vec_ref[...] * (1.0 / RC)
    d2 = jnp.sum(v * v, axis=1, keepdims=True)
    d = jnp.sqrt(d2)
    iszero = d == 0.0
    safe = jnp.where(iszero, 1.0, d)
    kpi = np.float32(np.pi) * (
        lax.broadcasted_iota(jnp.int32, (1, N_BASIS), 1) + 1).astype(
            jnp.float32)
    rb = SQRT2 * jnp.where(iszero, kpi, jnp.sin(kpi * safe) / safe)
    rb9 = jnp.concatenate([rb, jnp.ones_like(d)], axis=1)
    pre1 = _dot(rb9, w1a_ref[...]) + gs_ref[...] + gr_ref[...]
    pre1 = jnp.where(iszero, 0.0, pre1)
    x = _silu(pre1)
    x = _silu(_dot(x, we2_ref[...]))
    x = _silu(_dot(x, we3_ref[...]))
    x = _dot(x, we4_ref[...])
    d6 = d2 * d2 * d2
    d7 = d6 * d
    d8 = d7 * d
    env = jnp.where(d < 1.0, 1.0 - 28.0 * d6 + 48.0 * d7 - 21.0 * d8, 0.0)
    x = env * x
    rh = v / safe
    xx = rh[:, 0:1]
    yy = rh[:, 1:2]
    zz = rh[:, 2:3]
    Y = jnp.concatenate([
        jnp.ones_like(xx), SQRT3 * xx, SQRT3 * yy, SQRT3 * zz,
        SQRT15 * xx * yy, SQRT15 * yy * zz, (SQRT5 / 2.0) * (3.0 * zz * zz - 1.0),
        SQRT15 * xx * zz, (SQRT15 / 2.0) * (xx * xx - yy * yy)
    ], axis=1)
    x1_ref[...] = x
    ye = _dot(Y, rmat_ref[...])
    wy = ye * _dot(x, wenv9_ref[...])
    wy_ref[...] = jnp.concatenate(
        [wy, jnp.zeros((wy.shape[0], W9 - 9 * C), jnp.float32)], axis=1)
    geom_ref[...] = jnp.concatenate(
        [env, Y, jnp.zeros_like(Y[:, :6])], axis=1)


def _pass2_body(x1_ref, a_ref, geom_ref, wcomb_ref, wlat1_ref, wlat2e_ref,
                wall_ref, rmat_ref, tmat_ref, x2_ref, wy_ref, vvt_ref):
    x1 = x1_ref[...]
    g = geom_ref[...]
    env = g[:, 0:1]
    ye = _dot(g[:, 1:10], rmat_ref[...])
    wfull = _dot(x1, wcomb_ref[...])
    s = wfull[:, 0:C]
    wv3 = wfull[:, C:4 * C]
    wt5 = wfull[:, 4 * C:9 * C]
    x1we = wfull[:, 9 * C:18 * C]
    vvgw = ye[:, C:4 * C] * wv3
    tgw = ye[:, 4 * C:9 * C] * wt5
    A = a_ref[...][:, 0:9 * C] * INV_SQRT_ANN
    a0 = A[:, 0:C]
    a1w = A[:, C:4 * C]
    a2w = A[:, 4 * C:9 * C]
    pv = a1w * vvgw
    pt = a2w * tgw
    p00 = a0 * s
    p11 = (pv[:, 0:C] + pv[:, C:2 * C] + pv[:, 2 * C:3 * C]) * INV_SQRT3
    p22 = (pt[:, 0:C] + pt[:, C:2 * C] + pt[:, 2 * C:3 * C]
           + pt[:, 3 * C:4 * C] + pt[:, 4 * C:5 * C]) * INV_SQRT5
    xl = jnp.concatenate([x1, p00, p11, p22], axis=1)
    h = _silu(_dot(xl, wlat1_ref[...]))
    yz = _dot(h, wlat2e_ref[...])
    x2 = (x1 + (0.5 * env) * yz[:, 0:H]) * INV_SQRT125
    x2_ref[...] = x2
    wew = (x1we + (0.5 * env) * yz[:, H:H + 9 * C]) * INV_SQRT125
    wy = ye * wew
    wy_ref[...] = jnp.concatenate(
        [wy, jnp.zeros((wy.shape[0], W9 - 9 * C), jnp.float32)], axis=1)
    ts = _dot(jnp.concatenate([a0, s], axis=1), tmat_ref[...])
    a0_3 = ts[:, 0:3 * C]
    a0_5 = ts[:, 3 * C:8 * C]
    s3 = ts[:, 8 * C:11 * C]
    s5 = ts[:, 11 * C:16 * C]
    ax, ay, az = a1w[:, 0:C], a1w[:, C:2 * C], a1w[:, 2 * C:3 * C]
    bx, by, bz = vvgw[:, 0:C], vvgw[:, C:2 * C], vvgw[:, 2 * C:3 * C]
    r11w = jnp.concatenate(
        [S2 * (ax * by + ay * bx),
         S2 * (ay * bz + az * by),
         S6 * (2.0 * az * bz - ax * bx - ay * by),
         S2 * (ax * bz + az * bx),
         S2 * (ax * bx - ay * by)], axis=1)
    xall = jnp.concatenate(
        [a0_3 * vvgw, a1w * s3, a0_5 * tgw, a2w * s5, r11w], axis=1)
    vvt_ref[...] = _dot(xall, wall_ref[...])


def _pass3_body(x2_ref, a_ref, vvt_ref, geom_ref, wlat1r_ref, fo_ref,
                go_ref, o_ref):
    x2 = x2_ref[...]
    env = geom_ref[:, 0:1]
    A = a_ref[...][:, 0:9 * C] * INV_SQRT_ANN
    vvt = vvt_ref[...]
    pv = A[:, C:4 * C] * vvt[:, 0:3 * C]
    pt = A[:, 4 * C:9 * C] * vvt[:, 3 * C:8 * C]
    p11 = (pv[:, 0:C] + pv[:, C:2 * C] + pv[:, 2 * C:3 * C]) * INV_SQRT3
    p22 = (pt[:, 0:C] + pt[:, C:2 * C] + pt[:, 2 * C:3 * C]
           + pt[:, 3 * C:4 * C] + pt[:, 4 * C:5 * C]) * INV_SQRT5
    xl = jnp.concatenate([x2, p11, p22], axis=1)
    h = _silu(_dot(xl, wlat1r_ref[...]))
    o = (jnp.sum(x2 * fo_ref[...], axis=1, keepdims=True)
         + (0.5 * env) * jnp.sum(h * go_ref[...], axis=1, keepdims=True))
    o_ref[...] = o * INV_SQRT125


def _full(shape):
    return pl.BlockSpec(shape, lambda i: (0, 0))


def _blk(shape):
    return pl.BlockSpec(shape, lambda i: (i, 0))


BN = 400


def _node_proj(na, w1b, w1c):
    return pl.pallas_call(
        _node_proj_body,
        grid=(N_NODES // BN,),
        in_specs=[_blk((BN, D_FEAT)), _full((D_FEAT, 16)), _full((D_FEAT, 16))],
        out_specs=[_blk((BN, 16)), _blk((BN, 16))],
        out_shape=[jax.ShapeDtypeStruct((N_NODES, 16), _F32)] * 2,
    )(na, w1b, w1c)


def _pass1(vec, gs, gr, w1a, we2, we3, we4, wenv9, rmat):
    return pl.pallas_call(
        _pass1_body,
        grid=(N_BLK,),
        in_specs=[_blk((BE, 3)), _blk((BE, 16)), _blk((BE, 16)),
                  _full((N_BASIS + 1, 16)), _full((16, 32)), _full((32, 64)),
                  _full((64, 128)), _full((H, 9 * C)), _full((9, 9 * C))],
        out_specs=[_blk((BE, H)), _blk((BE, W9)), _blk((BE, 16))],
        out_shape=[jax.ShapeDtypeStruct((EP, H), _F32),
                   jax.ShapeDtypeStruct((EP, W9), _F32),
                   jax.ShapeDtypeStruct((EP, 16), _F32)],
    )(vec, gs, gr, w1a, we2, we3, we4, wenv9, rmat)


def _pass2(x1, a0, geom, wcomb, wlat1, wlat2e, wall, rmat, tmat):
    return pl.pallas_call(
        _pass2_body,
        grid=(N_BLK,),
        in_specs=[_blk((BE, H)), _blk((BE, W9)), _blk((BE, 16)),
                  _full((H, 18 * C)), _full((H + 3 * C, H)),
                  _full((H, H + 9 * C)), _full((21 * C, 8 * C)),
                  _full((9, 9 * C)), _full((2 * C, 16 * C))],
        out_specs=[_blk((BE, H)), _blk((BE, W9)), _blk((BE, 8 * C))],
        out_shape=[jax.ShapeDtypeStruct((EP, H), _F32),
                   jax.ShapeDtypeStruct((EP, W9), _F32),
                   jax.ShapeDtypeStruct((EP, 8 * C), _F32)],
    )(x1, a0, geom, wcomb, wlat1, wlat2e, wall, rmat, tmat)


def _pass3(x2, a1, vvt, geom, wlat1r, fo, go):
    return pl.pallas_call(
        _pass3_body,
        grid=(N_BLK,),
        in_specs=[_blk((BE, H)), _blk((BE, W9)), _blk((BE, 8 * C)),
                  _blk((BE, 16)), _full((H + 2 * C, H)), _full((1, H)),
                  _full((1, H))],
        out_specs=_blk((BE, 1)),
        out_shape=jax.ShapeDtypeStruct((EP, 1), _F32),
    )(x2, a1, vvt, geom, wlat1r, fo, go)


NW = 32
CH = 128
EPW = EP // NW
EPT = EP // 16
W9 = 3 * H
NHALF = N_NODES // 2
NSH = 5120
NROWT = NSH // 16


@functools.cache
def _sc_kernels():
    mesh = plsc.VectorSubcoreMesh(core_axis_name="c", subcore_axis_name="s",
                                  num_cores=2, num_subcores=16)
    params = pltpu.CompilerParams(use_tc_tiling_on_sc=False)

    @functools.partial(
        pl.kernel,
        out_type=[jax.ShapeDtypeStruct((EP, 16), _F32),
                  jax.ShapeDtypeStruct((EP, 16), _F32)],
        mesh=mesh,
        compiler_params=params,
        scratch_types=[
            pltpu.VMEM((CH,), jnp.int32),
            pltpu.VMEM((CH,), jnp.int32),
            pltpu.VMEM((CH, 16), _F32),
            pltpu.VMEM((CH, 16), _F32),
            pltpu.SemaphoreType.DMA,
            pltpu.SemaphoreType.DMA,
        ],
    )
    def sc_gather(ps_hbm, pr_hbm, snd_hbm, rcv_hbm, outs_hbm, outr_hbm,
                  idxs_v, idxr_v, rows_s, rows_r, sem_s, sem_r):
        wid = lax.axis_index("s") * 2 + lax.axis_index("c")
        base = wid * EPW

        @pl.loop(0, EPW // CH)
        def _(i):
            e0 = base + i * CH
            pltpu.sync_copy(snd_hbm.at[pl.ds(e0, CH)], idxs_v)
            pltpu.sync_copy(rcv_hbm.at[pl.ds(e0, CH)], idxr_v)
            cs = pltpu.async_copy(ps_hbm.at[idxs_v], rows_s, sem_s)
            cr = pltpu.async_copy(pr_hbm.at[idxr_v], rows_r, sem_r)
            cs.wait()
            cr.wait()
            pltpu.sync_copy(rows_s, outs_hbm.at[pl.ds(e0, CH)])
            pltpu.sync_copy(rows_r, outr_hbm.at[pl.ds(e0, CH)])

    tiled = pltpu.CompilerParams(use_tc_tiling_on_sc=True)

    @functools.partial(
        pl.kernel,
        out_type=jax.ShapeDtypeStruct((2 * NSH, W9), _F32),
        mesh=mesh,
        compiler_params=tiled,
        scratch_types=[
            pltpu.VMEM((8, CH), jnp.int32),
            pltpu.VMEM((CH, H), _F32),
            pltpu.VMEM_SHARED((NSH, H), _F32),
        ],
    )
    def sc_scatter(wy_hbm, snd3d_hbm, zrows_hbm, tab_hbm, idx_v, rows_v,
                   acc_sh):
        cid = lax.axis_index("c")
        sid = lax.axis_index("s")
        lo = cid * NHALF
        base = sid * EPT
        lane = lax.iota(jnp.int32, 16)
        for ph in range(3):
            pltpu.sync_copy(zrows_hbm.at[pl.ds(sid * NROWT, NROWT)],
                            acc_sh.at[pl.ds(sid * NROWT, NROWT)])
            plsc.subcore_barrier()

            @pl.loop(0, EPT // (8 * CH))
            def _(i):
                pltpu.sync_copy(snd3d_hbm.at[base // (8 * CH) + i], idx_v)
                for j in range(8):
                    for t in range(8):
                        v = idx_v[j, pl.ds(16 * t, 16)] - lo
                        oob = (v < 0) | (v >= NHALF)
                        trash = NHALF + ((lane + 16 * t + j) & 63)
                        idx_v[j, pl.ds(16 * t, 16)] = jnp.where(oob, trash, v)
                for j in range(8):
                    e0 = base + i * 8 * CH + j * CH
                    pltpu.sync_copy(
                        wy_hbm.at[pl.ds(e0, CH), pl.ds(ph * H, H)], rows_v)
                    pltpu.sync_copy(rows_v, acc_sh.at[idx_v.at[j]], add=True)

            plsc.subcore_barrier()
            pltpu.sync_copy(
                acc_sh.at[pl.ds(sid * NROWT, NROWT)],
                tab_hbm.at[pl.ds(cid * NSH + sid * NROWT, NROWT),
                           pl.ds(ph * H, H)])
            plsc.subcore_barrier()

    @functools.partial(
        pl.kernel,
        out_type=jax.ShapeDtypeStruct((EP, W9), _F32),
        mesh=mesh,
        compiler_params=tiled,
        scratch_types=[
            pltpu.VMEM((8, CH), jnp.int32),
            pltpu.VMEM((CH, W9), _F32),
            pltpu.SemaphoreType.DMA,
        ],
    )
    def sc_gatherback(tab_hbm, snd3d_hbm, out_hbm, idx_v, rows_v, sem):
        wid = lax.axis_index("s") * 2 + lax.axis_index("c")
        base = wid * EPW

        @pl.loop(0, EPW // (8 * CH))
        def _(i):
            pltpu.sync_copy(snd3d_hbm.at[base // (8 * CH) + i], idx_v)
            for j in range(8):
                for t in range(8):
                    v = idx_v[j, pl.ds(16 * t, 16)]
                    idx_v[j, pl.ds(16 * t, 16)] = jnp.where(
                        v >= NHALF, v + (NSH - NHALF), v)
            for j in range(8):
                e0 = base + i * 8 * CH + j * CH
                pltpu.async_copy(tab_hbm.at[idx_v.at[j]], rows_v, sem).wait()
                pltpu.sync_copy(rows_v, out_hbm.at[pl.ds(e0, CH)])

    return sc_gather, sc_scatter, sc_gatherback


def kernel(node_attrs, vectors, senders, receivers, W_e1, W_e2, W_e3, W_e4,
           W_w0, W_env_0, W_lat1_0, W_lat2_0, W_l1_0, W_l2_0,
           W_env_1, W_lat1_1, W_lat2_1, W_l1_1, W_l2_1, W_f, W_out):
    senders = senders.astype(jnp.int32)
    receivers = receivers.astype(jnp.int32)
    pad_idx = jnp.arange(PAD, dtype=jnp.int32) * (N_NODES // PAD)
    snd_p = jnp.concatenate([senders, pad_idx])
    rcv_p = jnp.concatenate([receivers, pad_idx])
    vec_p = jnp.concatenate(
        [vectors.astype(_F32), jnp.zeros((PAD, 3), _F32)])

    sc_gather, sc_scatter, sc_gatherback = _sc_kernels()
    ps, pr = _node_proj(node_attrs.astype(_F32),
                        W_e1[N_BASIS:N_BASIS + D_FEAT],
                        W_e1[N_BASIS + D_FEAT:])
    gs, gr = sc_gather(ps, pr, snd_p, rcv_p)
    w1a = jnp.concatenate([
        W_e1[:N_BASIS] * jnp.asarray(INVSIG_NP.T),
        -jnp.asarray(MU_NP * INVSIG_NP) @ W_e1[:N_BASIS],
    ], axis=0)
    rmat = jnp.asarray(R_NP)
    tmat = jnp.asarray(TMAT_NP)
    wenv9 = jnp.tile(W_env_0, (1, 9))
    x1, wy0, geom = _pass1(vec_p, gs, gr, w1a, W_e2, W_e3, W_e4, wenv9, rmat)
    snd3d = snd_p.reshape(EP // (8 * CH), 8, CH)
    zrows = jnp.zeros((NSH, H), _F32)
    a0 = sc_gatherback(sc_scatter(wy0, snd3d, zrows), snd3d)
    wcomb = jnp.concatenate([
        W_w0[:, 0:C], jnp.tile(W_w0[:, C:2 * C], (1, 3)),
        jnp.tile(W_w0[:, 2 * C:3 * C], (1, 5)), jnp.tile(W_env_1, (1, 9)),
    ], axis=1)
    wlat2e = jnp.concatenate(
        [W_lat2_0, jnp.tile(W_lat2_0 @ W_env_1, (1, 9))], axis=1)
    wall = jnp.zeros((21 * C, 8 * C), _F32)
    for k in range(3):
        wall = wall.at[C * k:C * (k + 1), C * k:C * (k + 1)].set(
            W_l1_0[:C])
        wall = wall.at[3 * C + C * k:3 * C + C * (k + 1),
                       C * k:C * (k + 1)].set(W_l1_0[C:])
    for m in range(5):
        wall = wall.at[6 * C + C * m:6 * C + C * (m + 1),
                       3 * C + C * m:3 * C + C * (m + 1)].set(
            W_l2_0[:C])
        wall = wall.at[11 * C + C * m:11 * C + C * (m + 1),
                       3 * C + C * m:3 * C + C * (m + 1)].set(
            W_l2_0[C:2 * C])
        wall = wall.at[16 * C + C * m:16 * C + C * (m + 1),
                       3 * C + C * m:3 * C + C * (m + 1)].set(
            W_l2_0[2 * C:])
    x2, wy1, vvt = _pass2(x1, a0, geom, wcomb, W_lat1_0, wlat2e, wall,
                          rmat, tmat)
    a1 = sc_gatherback(sc_scatter(wy1, snd3d, zrows), snd3d)
    fo = W_f @ W_out[:H]
    go = W_lat2_1 @ fo
    wlat1r = jnp.concatenate([W_lat1_1[:H], W_lat1_1[H + C:]], axis=0)
    out = _pass3(x2, a1, vvt, geom, wlat1r, fo.reshape(1, H), go.reshape(1, H))
    return out[:N_EDGES]


# speedup vs baseline: 12.2658x; 1.0861x over previous
import functools

import numpy as np
import jax
import jax.numpy as jnp
from jax import lax
from jax.experimental import pallas as pl
from jax.experimental.pallas import tpu as pltpu
from jax.experimental.pallas import tpu_sc as plsc

N_NODES = 10000
N_EDGES = 160000
D_FEAT = 128
N_BASIS = 8
C = 32
P = 6
RC = 4.0
ANN = 32.0
H = 128

EP = 163840
BE = 512
N_BLK = EP // BE
PAD = EP - N_EDGES

_trapz = getattr(np, 'trapezoid', None) or np.trapz


def _bessel_np(r, n):
    k = np.arange(1, n + 1)[None, :]
    r = r[:, None]
    safe = np.where(r == 0.0, 1.0, r)
    return np.sqrt(2.0) * np.where(r == 0.0, k * np.pi, np.sin(k * np.pi * safe) / safe)


_r = np.linspace(0.0, 1.0, 1000)
_b = _bessel_np(_r, N_BASIS)
_MU = _trapz(_b, _r, axis=0)
_SIG = _trapz((_b - _MU) ** 2, _r, axis=0) ** 0.5
MU_NP = np.asarray(_MU, dtype=np.float32)[None, :]
INVSIG_NP = np.asarray(1.0 / _SIG, dtype=np.float32)[None, :]

SQRT2 = float(np.sqrt(2.0))
SQRT3 = float(np.sqrt(3.0))
SQRT5 = float(np.sqrt(5.0))
SQRT15 = float(np.sqrt(15.0))
INV_SQRT3 = float(1.0 / np.sqrt(3.0))
INV_SQRT5 = float(1.0 / np.sqrt(5.0))
INV_SQRT_ANN = float(1.0 / np.sqrt(ANN))
INV_SQRT125 = float(1.0 / np.sqrt(1.25))
S2 = float(1.0 / np.sqrt(2.0))
S6 = float(1.0 / np.sqrt(6.0))

_F32 = jnp.float32

R_NP = np.zeros((9, 9 * C), dtype=np.float32)
for _j in range(9):
    R_NP[_j, C * _j:C * (_j + 1)] = 1.0
TMAT_NP = np.zeros((2 * C, 16 * C), dtype=np.float32)
for _k in range(3):
    TMAT_NP[np.arange(C), C * _k + np.arange(C)] = 1.0
    TMAT_NP[C + np.arange(C), 8 * C + C * _k + np.arange(C)] = 1.0
for _m in range(5):
    TMAT_NP[np.arange(C), 3 * C + C * _m + np.arange(C)] = 1.0
    TMAT_NP[C + np.arange(C), 11 * C + C * _m + np.arange(C)] = 1.0


def _silu(x):
    return x * jax.nn.sigmoid(x)


def _dot(a, b):
    return jnp.dot(a, b, preferred_element_type=jnp.float32)


def _node_proj_body(na_ref, w1b_ref, w1c_ref, ps_ref, pr_ref):
    na = na_ref[...]
    ps_ref[...] = _dot(na, w1b_ref[...])
    pr_ref[...] = _dot(na, w1c_ref[...])


def _pass1_body(vec_ref, gs_ref, gr_ref, w1a_ref, we2_ref, we3_ref, we4_ref,
                wenv9_ref, rmat_ref, x1_ref, wy_ref, geom_ref):
    v = vec_ref[...] * (1.0 / RC)
    d2 = jnp.sum(v * v, axis=1, keepdims=True)
    d = jnp.sqrt(d2)
    iszero = d == 0.0
    safe = jnp.where(iszero, 1.0, d)
    kpi = np.float32(np.pi) * (
        lax.broadcasted_iota(jnp.int32, (1, N_BASIS), 1) + 1).astype(
            jnp.float32)
    rb = SQRT2 * jnp.where(iszero, kpi, jnp.sin(kpi * safe) / safe)
    rb9 = jnp.concatenate([rb, jnp.ones_like(d)], axis=1)
    pre1 = _dot(rb9, w1a_ref[...]) + gs_ref[...] + gr_ref[...]
    pre1 = jnp.where(iszero, 0.0, pre1)
    x = _silu(pre1)
    x = _silu(_dot(x, we2_ref[...]))
    x = _silu(_dot(x, we3_ref[...]))
    x = _dot(x, we4_ref[...])
    d6 = d2 * d2 * d2
    d7 = d6 * d
    d8 = d7 * d
    env = jnp.where(d < 1.0, 1.0 - 28.0 * d6 + 48.0 * d7 - 21.0 * d8, 0.0)
    x = env * x
    rh = v / safe
    xx = rh[:, 0:1]
    yy = rh[:, 1:2]
    zz = rh[:, 2:3]
    Y = jnp.concatenate([
        jnp.ones_like(xx), SQRT3 * xx, SQRT3 * yy, SQRT3 * zz,
        SQRT15 * xx * yy, SQRT15 * yy * zz, (SQRT5 / 2.0) * (3.0 * zz * zz - 1.0),
        SQRT15 * xx * zz, (SQRT15 / 2.0) * (xx * xx - yy * yy)
    ], axis=1)
    x1_ref[...] = x
    ye = _dot(Y, rmat_ref[...])
    wy = ye * _dot(x, wenv9_ref[...])
    wy_ref[...] = jnp.concatenate(
        [wy, jnp.zeros((wy.shape[0], W9 - 9 * C), jnp.float32)], axis=1)
    geom_ref[...] = jnp.concatenate(
        [env, Y, jnp.zeros_like(Y[:, :6])], axis=1)


def _pass2_body(x1_ref, a_ref, geom_ref, wcomb_ref, wlat1_ref, wlat2e_ref,
                wall_ref, rmat_ref, tmat_ref, x2_ref, wy_ref, vvt_ref):
    x1 = x1_ref[...]
    g = geom_ref[...]
    env = g[:, 0:1]
    ye = _dot(g[:, 1:10], rmat_ref[...])
    wfull = _dot(x1, wcomb_ref[...])
    s = wfull[:, 0:C]
    wv3 = wfull[:, C:4 * C]
    wt5 = wfull[:, 4 * C:9 * C]
    x1we = wfull[:, 9 * C:18 * C]
    vvgw = ye[:, C:4 * C] * wv3
    tgw = ye[:, 4 * C:9 * C] * wt5
    A = a_ref[...][:, 0:9 * C] * INV_SQRT_ANN
    a0 = A[:, 0:C]
    a1w = A[:, C:4 * C]
    a2w = A[:, 4 * C:9 * C]
    pv = a1w * vvgw
    pt = a2w * tgw
    p00 = a0 * s
    p11 = (pv[:, 0:C] + pv[:, C:2 * C] + pv[:, 2 * C:3 * C]) * INV_SQRT3
    p22 = (pt[:, 0:C] + pt[:, C:2 * C] + pt[:, 2 * C:3 * C]
           + pt[:, 3 * C:4 * C] + pt[:, 4 * C:5 * C]) * INV_SQRT5
    xl = jnp.concatenate([x1, p00, p11, p22], axis=1)
    h = _silu(_dot(xl, wlat1_ref[...]))
    yz = _dot(h, wlat2e_ref[...])
    x2 = (x1 + (0.5 * env) * yz[:, 0:H]) * INV_SQRT125
    x2_ref[...] = x2
    wew = (x1we + (0.5 * env) * yz[:, H:H + 9 * C]) * INV_SQRT125
    wy = ye * wew
    wy_ref[...] = jnp.concatenate(
        [wy, jnp.zeros((wy.shape[0], W9 - 9 * C), jnp.float32)], axis=1)
    ts = _dot(jnp.concatenate([a0, s], axis=1), tmat_ref[...])
    a0_3 = ts[:, 0:3 * C]
    a0_5 = ts[:, 3 * C:8 * C]
    s3 = ts[:, 8 * C:11 * C]
    s5 = ts[:, 11 * C:16 * C]
    ax, ay, az = a1w[:, 0:C], a1w[:, C:2 * C], a1w[:, 2 * C:3 * C]
    bx, by, bz = vvgw[:, 0:C], vvgw[:, C:2 * C], vvgw[:, 2 * C:3 * C]
    r11w = jnp.concatenate(
        [S2 * (ax * by + ay * bx),
         S2 * (ay * bz + az * by),
         S6 * (2.0 * az * bz - ax * bx - ay * by),
         S2 * (ax * bz + az * bx),
         S2 * (ax * bx - ay * by)], axis=1)
    xall = jnp.concatenate(
        [a0_3 * vvgw, a1w * s3, a0_5 * tgw, a2w * s5, r11w], axis=1)
    vvt_ref[...] = _dot(xall, wall_ref[...])


def _pass3_body(x2_ref, a_ref, vvt_ref, geom_ref, wlat1r_ref, fo_ref,
                go_ref, o_ref):
    x2 = x2_ref[...]
    env = geom_ref[:, 0:1]
    A = a_ref[...][:, 0:9 * C] * INV_SQRT_ANN
    vvt = vvt_ref[...]
    pv = A[:, C:4 * C] * vvt[:, 0:3 * C]
    pt = A[:, 4 * C:9 * C] * vvt[:, 3 * C:8 * C]
    p11 = (pv[:, 0:C] + pv[:, C:2 * C] + pv[:, 2 * C:3 * C]) * INV_SQRT3
    p22 = (pt[:, 0:C] + pt[:, C:2 * C] + pt[:, 2 * C:3 * C]
           + pt[:, 3 * C:4 * C] + pt[:, 4 * C:5 * C]) * INV_SQRT5
    xl = jnp.concatenate([x2, p11, p22], axis=1)
    h = _silu(_dot(xl, wlat1r_ref[...]))
    o = (jnp.sum(x2 * fo_ref[...], axis=1, keepdims=True)
         + (0.5 * env) * jnp.sum(h * go_ref[...], axis=1, keepdims=True))
    o_ref[...] = o * INV_SQRT125


def _full(shape):
    return pl.BlockSpec(shape, lambda i: (0, 0))


def _blk(shape):
    return pl.BlockSpec(shape, lambda i: (i, 0))


BN = 400


def _node_proj(na, w1b, w1c):
    return pl.pallas_call(
        _node_proj_body,
        grid=(N_NODES // BN,),
        in_specs=[_blk((BN, D_FEAT)), _full((D_FEAT, 16)), _full((D_FEAT, 16))],
        out_specs=[_blk((BN, 16)), _blk((BN, 16))],
        out_shape=[jax.ShapeDtypeStruct((N_NODES, 16), _F32)] * 2,
    )(na, w1b, w1c)


def _pass1(vec, gs, gr, w1a, we2, we3, we4, wenv9, rmat):
    return pl.pallas_call(
        _pass1_body,
        grid=(N_BLK,),
        in_specs=[_blk((BE, 3)), _blk((BE, 16)), _blk((BE, 16)),
                  _full((N_BASIS + 1, 16)), _full((16, 32)), _full((32, 64)),
                  _full((64, 128)), _full((H, 9 * C)), _full((9, 9 * C))],
        out_specs=[_blk((BE, H)), _blk((BE, W9)), _blk((BE, 16))],
        out_shape=[jax.ShapeDtypeStruct((EP, H), _F32),
                   jax.ShapeDtypeStruct((EP, W9), _F32),
                   jax.ShapeDtypeStruct((EP, 16), _F32)],
    )(vec, gs, gr, w1a, we2, we3, we4, wenv9, rmat)


def _pass2(x1, a0, geom, wcomb, wlat1, wlat2e, wall, rmat, tmat):
    return pl.pallas_call(
        _pass2_body,
        grid=(N_BLK,),
        in_specs=[_blk((BE, H)), _blk((BE, W9)), _blk((BE, 16)),
                  _full((H, 18 * C)), _full((H + 3 * C, H)),
                  _full((H, H + 9 * C)), _full((21 * C, 8 * C)),
                  _full((9, 9 * C)), _full((2 * C, 16 * C))],
        out_specs=[_blk((BE, H)), _blk((BE, W9)), _blk((BE, 8 * C))],
        out_shape=[jax.ShapeDtypeStruct((EP, H), _F32),
                   jax.ShapeDtypeStruct((EP, W9), _F32),
                   jax.ShapeDtypeStruct((EP, 8 * C), _F32)],
    )(x1, a0, geom, wcomb, wlat1, wlat2e, wall, rmat, tmat)


def _pass3(x2, a1, vvt, geom, wlat1r, fo, go):
    return pl.pallas_call(
        _pass3_body,
        grid=(N_BLK,),
        in_specs=[_blk((BE, H)), _blk((BE, W9)), _blk((BE, 8 * C)),
                  _blk((BE, 16)), _full((H + 2 * C, H)), _full((1, H)),
                  _full((1, H))],
        out_specs=_blk((BE, 1)),
        out_shape=jax.ShapeDtypeStruct((EP, 1), _F32),
    )(x2, a1, vvt, geom, wlat1r, fo, go)


NW = 32
CH = 128
EPW = EP // NW
EPT = EP // 16
W9 = 3 * H
NHALF = N_NODES // 2
NSH = 5120
NROWT = NSH // 16


@functools.cache
def _sc_kernels():
    mesh = plsc.VectorSubcoreMesh(core_axis_name="c", subcore_axis_name="s",
                                  num_cores=2, num_subcores=16)
    params = pltpu.CompilerParams(use_tc_tiling_on_sc=False)

    @functools.partial(
        pl.kernel,
        out_type=[jax.ShapeDtypeStruct((EP, 16), _F32),
                  jax.ShapeDtypeStruct((EP, 16), _F32)],
        mesh=mesh,
        compiler_params=params,
        scratch_types=[
            pltpu.VMEM((CH,), jnp.int32),
            pltpu.VMEM((CH,), jnp.int32),
            pltpu.VMEM((CH, 16), _F32),
            pltpu.VMEM((CH, 16), _F32),
            pltpu.SemaphoreType.DMA,
            pltpu.SemaphoreType.DMA,
        ],
    )
    def sc_gather(ps_hbm, pr_hbm, snd_hbm, rcv_hbm, outs_hbm, outr_hbm,
                  idxs_v, idxr_v, rows_s, rows_r, sem_s, sem_r):
        wid = lax.axis_index("s") * 2 + lax.axis_index("c")
        base = wid * EPW

        @pl.loop(0, EPW // CH)
        def _(i):
            e0 = base + i * CH
            pltpu.sync_copy(snd_hbm.at[pl.ds(e0, CH)], idxs_v)
            pltpu.sync_copy(rcv_hbm.at[pl.ds(e0, CH)], idxr_v)
            cs = pltpu.async_copy(ps_hbm.at[idxs_v], rows_s, sem_s)
            cr = pltpu.async_copy(pr_hbm.at[idxr_v], rows_r, sem_r)
            cs.wait()
            cr.wait()
            pltpu.sync_copy(rows_s, outs_hbm.at[pl.ds(e0, CH)])
            pltpu.sync_copy(rows_r, outr_hbm.at[pl.ds(e0, CH)])

    tiled = pltpu.CompilerParams(use_tc_tiling_on_sc=True)

    @functools.partial(
        pl.kernel,
        out_type=jax.ShapeDtypeStruct((2 * NSH, W9), _F32),
        mesh=mesh,
        compiler_params=tiled,
        scratch_types=[
            pltpu.VMEM((8, CH), jnp.int32),
            pltpu.VMEM((2, CH, H), _F32),
            pltpu.VMEM_SHARED((NSH, H), _F32),
            pltpu.SemaphoreType.DMA,
            pltpu.SemaphoreType.DMA,
        ],
    )
    def sc_scatter(wy_hbm, snd3d_hbm, zrows_hbm, tab_hbm, idx_v, rows_v,
                   acc_sh, sem0, sem1):
        cid = lax.axis_index("c")
        sid = lax.axis_index("s")
        lo = cid * NHALF
        base = sid * EPT
        lane = lax.iota(jnp.int32, 16)
        for ph in range(3):
            pltpu.sync_copy(zrows_hbm.at[pl.ds(sid * NROWT, NROWT)],
                            acc_sh.at[pl.ds(sid * NROWT, NROWT)])
            plsc.subcore_barrier()

            @pl.loop(0, EPT // (8 * CH))
            def _(i):
                pltpu.sync_copy(snd3d_hbm.at[base // (8 * CH) + i], idx_v)
                for j in range(8):
                    for t in range(8):
                        v = idx_v[j, pl.ds(16 * t, 16)] - lo
                        oob = (v < 0) | (v >= NHALF)
                        trash = NHALF + ((lane + 16 * t + j) & 63)
                        idx_v[j, pl.ds(16 * t, 16)] = jnp.where(oob, trash, v)
                sems = [sem0, sem1]
                wps = [None, None]
                for j in range(8):
                    e0 = base + i * 8 * CH + j * CH
                    if wps[j % 2] is not None:
                        wps[j % 2].wait()
                    pltpu.sync_copy(
                        wy_hbm.at[pl.ds(e0, CH), pl.ds(ph * H, H)],
                        rows_v.at[j % 2])
                    wps[j % 2] = pltpu.async_copy(
                        rows_v.at[j % 2], acc_sh.at[idx_v.at[j]],
                        sems[j % 2], add=True)
                wps[0].wait()
                wps[1].wait()

            plsc.subcore_barrier()
            pltpu.sync_copy(
                acc_sh.at[pl.ds(sid * NROWT, NROWT)],
                tab_hbm.at[pl.ds(cid * NSH + sid * NROWT, NROWT),
                           pl.ds(ph * H, H)])
            plsc.subcore_barrier()

    @functools.partial(
        pl.kernel,
        out_type=jax.ShapeDtypeStruct((EP, W9), _F32),
        mesh=mesh,
        compiler_params=tiled,
        scratch_types=[
            pltpu.VMEM((8, CH), jnp.int32),
            pltpu.VMEM((2, CH, W9), _F32),
            pltpu.SemaphoreType.DMA,
            pltpu.SemaphoreType.DMA,
            pltpu.SemaphoreType.DMA,
            pltpu.SemaphoreType.DMA,
        ],
    )
    def sc_gatherback(tab_hbm, snd3d_hbm, out_hbm, idx_v, rows_v,
                      gs0, gs1, ws0, ws1):
        wid = lax.axis_index("s") * 2 + lax.axis_index("c")
        base = wid * EPW
        gsems = [gs0, gs1]
        wsems = [ws0, ws1]

        @pl.loop(0, EPW // (8 * CH))
        def _(i):
            pltpu.sync_copy(snd3d_hbm.at[base // (8 * CH) + i], idx_v)
            for j in range(8):
                for t in range(8):
                    v = idx_v[j, pl.ds(16 * t, 16)]
                    idx_v[j, pl.ds(16 * t, 16)] = jnp.where(
                        v >= NHALF, v + (NSH - NHALF), v)
            cg = [None] * 8
            ww = [None] * 8
            cg[0] = pltpu.async_copy(tab_hbm.at[idx_v.at[0]], rows_v.at[0],
                                     gsems[0])
            for j in range(8):
                e0 = base + i * 8 * CH + j * CH
                cg[j].wait()
                ww[j] = pltpu.async_copy(rows_v.at[j % 2],
                                         out_hbm.at[pl.ds(e0, CH)],
                                         wsems[j % 2])
                if j + 1 < 8:
                    if j >= 1:
                        ww[j - 1].wait()
                    cg[j + 1] = pltpu.async_copy(
                        tab_hbm.at[idx_v.at[j + 1]],
                        rows_v.at[(j + 1) % 2], gsems[(j + 1) % 2])
            ww[6].wait()
            ww[7].wait()

    return sc_gather, sc_scatter, sc_gatherback


def kernel(node_attrs, vectors, senders, receivers, W_e1, W_e2, W_e3, W_e4,
           W_w0, W_env_0, W_lat1_0, W_lat2_0, W_l1_0, W_l2_0,
           W_env_1, W_lat1_1, W_lat2_1, W_l1_1, W_l2_1, W_f, W_out):
    senders = senders.astype(jnp.int32)
    receivers = receivers.astype(jnp.int32)
    pad_idx = jnp.arange(PAD, dtype=jnp.int32) * (N_NODES // PAD)
    snd_p = jnp.concatenate([senders, pad_idx])
    rcv_p = jnp.concatenate([receivers, pad_idx])
    vec_p = jnp.concatenate(
        [vectors.astype(_F32), jnp.zeros((PAD, 3), _F32)])

    sc_gather, sc_scatter, sc_gatherback = _sc_kernels()
    ps, pr = _node_proj(node_attrs.astype(_F32),
                        W_e1[N_BASIS:N_BASIS + D_FEAT],
                        W_e1[N_BASIS + D_FEAT:])
    gs, gr = sc_gather(ps, pr, snd_p, rcv_p)
    w1a = jnp.concatenate([
        W_e1[:N_BASIS] * jnp.asarray(INVSIG_NP.T),
        -jnp.asarray(MU_NP * INVSIG_NP) @ W_e1[:N_BASIS],
    ], axis=0)
    rmat = jnp.asarray(R_NP)
    tmat = jnp.asarray(TMAT_NP)
    wenv9 = jnp.tile(W_env_0, (1, 9))
    x1, wy0, geom = _pass1(vec_p, gs, gr, w1a, W_e2, W_e3, W_e4, wenv9, rmat)
    snd3d = snd_p.reshape(EP // (8 * CH), 8, CH)
    zrows = jnp.zeros((NSH, H), _F32)
    a0 = sc_gatherback(sc_scatter(wy0, snd3d, zrows), snd3d)
    wcomb = jnp.concatenate([
        W_w0[:, 0:C], jnp.tile(W_w0[:, C:2 * C], (1, 3)),
        jnp.tile(W_w0[:, 2 * C:3 * C], (1, 5)), jnp.tile(W_env_1, (1, 9)),
    ], axis=1)
    wlat2e = jnp.concatenate(
        [W_lat2_0, jnp.tile(W_lat2_0 @ W_env_1, (1, 9))], axis=1)
    wall = jnp.zeros((21 * C, 8 * C), _F32)
    for k in range(3):
        wall = wall.at[C * k:C * (k + 1), C * k:C * (k + 1)].set(
            W_l1_0[:C])
        wall = wall.at[3 * C + C * k:3 * C + C * (k + 1),
                       C * k:C * (k + 1)].set(W_l1_0[C:])
    for m in range(5):
        wall = wall.at[6 * C + C * m:6 * C + C * (m + 1),
                       3 * C + C * m:3 * C + C * (m + 1)].set(
            W_l2_0[:C])
        wall = wall.at[11 * C + C * m:11 * C + C * (m + 1),
                       3 * C + C * m:3 * C + C * (m + 1)].set(
            W_l2_0[C:2 * C])
        wall = wall.at[16 * C + C * m:16 * C + C * (m + 1),
                       3 * C + C * m:3 * C + C * (m + 1)].set(
            W_l2_0[2 * C:])
    x2, wy1, vvt = _pass2(x1, a0, geom, wcomb, W_lat1_0, wlat2e, wall,
                          rmat, tmat)
    a1 = sc_gatherback(sc_scatter(wy1, snd3d, zrows), snd3d)
    fo = W_f @ W_out[:H]
    go = W_lat2_1 @ fo
    wlat1r = jnp.concatenate([W_lat1_1[:H], W_lat1_1[H + C:]], axis=0)
    out = _pass3(x2, a1, vvt, geom, wlat1r, fo.reshape(1, H), go.reshape(1, H))
    return out[:N_EDGES]


# speedup vs baseline: 13.1228x; 1.0699x over previous
import functools

import numpy as np
import jax
import jax.numpy as jnp
from jax import lax
from jax.experimental import pallas as pl
from jax.experimental.pallas import tpu as pltpu
from jax.experimental.pallas import tpu_sc as plsc

N_NODES = 10000
N_EDGES = 160000
D_FEAT = 128
N_BASIS = 8
C = 32
P = 6
RC = 4.0
ANN = 32.0
H = 128

EP = 163840
BE = 1024
N_BLK = EP // BE
PAD = EP - N_EDGES

_trapz = getattr(np, 'trapezoid', None) or np.trapz


def _bessel_np(r, n):
    k = np.arange(1, n + 1)[None, :]
    r = r[:, None]
    safe = np.where(r == 0.0, 1.0, r)
    return np.sqrt(2.0) * np.where(r == 0.0, k * np.pi, np.sin(k * np.pi * safe) / safe)


_r = np.linspace(0.0, 1.0, 1000)
_b = _bessel_np(_r, N_BASIS)
_MU = _trapz(_b, _r, axis=0)
_SIG = _trapz((_b - _MU) ** 2, _r, axis=0) ** 0.5
MU_NP = np.asarray(_MU, dtype=np.float32)[None, :]
INVSIG_NP = np.asarray(1.0 / _SIG, dtype=np.float32)[None, :]

SQRT2 = float(np.sqrt(2.0))
SQRT3 = float(np.sqrt(3.0))
SQRT5 = float(np.sqrt(5.0))
SQRT15 = float(np.sqrt(15.0))
INV_SQRT3 = float(1.0 / np.sqrt(3.0))
INV_SQRT5 = float(1.0 / np.sqrt(5.0))
INV_SQRT_ANN = float(1.0 / np.sqrt(ANN))
INV_SQRT125 = float(1.0 / np.sqrt(1.25))
S2 = float(1.0 / np.sqrt(2.0))
S6 = float(1.0 / np.sqrt(6.0))

_F32 = jnp.float32

R_NP = np.zeros((9, 9 * C), dtype=np.float32)
for _j in range(9):
    R_NP[_j, C * _j:C * (_j + 1)] = 1.0
TMAT_NP = np.zeros((2 * C, 16 * C), dtype=np.float32)
for _k in range(3):
    TMAT_NP[np.arange(C), C * _k + np.arange(C)] = 1.0
    TMAT_NP[C + np.arange(C), 8 * C + C * _k + np.arange(C)] = 1.0
for _m in range(5):
    TMAT_NP[np.arange(C), 3 * C + C * _m + np.arange(C)] = 1.0
    TMAT_NP[C + np.arange(C), 11 * C + C * _m + np.arange(C)] = 1.0


def _silu(x):
    return x * jax.nn.sigmoid(x)


def _dot(a, b):
    return jnp.dot(a, b, preferred_element_type=jnp.float32)


def _node_proj_body(na_ref, w1b_ref, w1c_ref, ps_ref, pr_ref):
    na = na_ref[...]
    ps_ref[...] = _dot(na, w1b_ref[...])
    pr_ref[...] = _dot(na, w1c_ref[...])


def _pass1_body(vec_ref, gs_ref, gr_ref, w1a_ref, we2_ref, we3_ref, we4_ref,
                wenv9_ref, rmat_ref, x1_ref, wy_ref, geom_ref):
    v = vec_ref[...] * (1.0 / RC)
    d2 = jnp.sum(v * v, axis=1, keepdims=True)
    d = jnp.sqrt(d2)
    iszero = d == 0.0
    safe = jnp.where(iszero, 1.0, d)
    kpi = np.float32(np.pi) * (
        lax.broadcasted_iota(jnp.int32, (1, N_BASIS), 1) + 1).astype(
            jnp.float32)
    rb = SQRT2 * jnp.where(iszero, kpi, jnp.sin(kpi * safe) / safe)
    rb9 = jnp.concatenate([rb, jnp.ones_like(d)], axis=1)
    pre1 = _dot(rb9, w1a_ref[...]) + gs_ref[...] + gr_ref[...]
    pre1 = jnp.where(iszero, 0.0, pre1)
    x = _silu(pre1)
    x = _silu(_dot(x, we2_ref[...]))
    x = _silu(_dot(x, we3_ref[...]))
    x = _dot(x, we4_ref[...])
    d6 = d2 * d2 * d2
    d7 = d6 * d
    d8 = d7 * d
    env = jnp.where(d < 1.0, 1.0 - 28.0 * d6 + 48.0 * d7 - 21.0 * d8, 0.0)
    x = env * x
    rh = v / safe
    xx = rh[:, 0:1]
    yy = rh[:, 1:2]
    zz = rh[:, 2:3]
    Y = jnp.concatenate([
        jnp.ones_like(xx), SQRT3 * xx, SQRT3 * yy, SQRT3 * zz,
        SQRT15 * xx * yy, SQRT15 * yy * zz, (SQRT5 / 2.0) * (3.0 * zz * zz - 1.0),
        SQRT15 * xx * zz, (SQRT15 / 2.0) * (xx * xx - yy * yy)
    ], axis=1)
    x1_ref[...] = x
    ye = _dot(Y, rmat_ref[...])
    wy = ye * _dot(x, wenv9_ref[...])
    wy_ref[...] = jnp.concatenate(
        [wy, jnp.zeros((wy.shape[0], W9 - 9 * C), jnp.float32)], axis=1)
    geom_ref[...] = jnp.concatenate(
        [env, Y, jnp.zeros_like(Y[:, :6])], axis=1)


def _pass2_body(x1_ref, a_ref, geom_ref, wcomb_ref, wlat1_ref, wlat2e_ref,
                wall_ref, rmat_ref, tmat_ref, x2_ref, wy_ref, vvt_ref):
    x1 = x1_ref[...]
    g = geom_ref[...]
    env = g[:, 0:1]
    ye = _dot(g[:, 1:10], rmat_ref[...])
    wfull = _dot(x1, wcomb_ref[...])
    s = wfull[:, 0:C]
    wv3 = wfull[:, C:4 * C]
    wt5 = wfull[:, 4 * C:9 * C]
    x1we = wfull[:, 9 * C:18 * C]
    vvgw = ye[:, C:4 * C] * wv3
    tgw = ye[:, 4 * C:9 * C] * wt5
    A = a_ref[...][:, 0:9 * C] * INV_SQRT_ANN
    a0 = A[:, 0:C]
    a1w = A[:, C:4 * C]
    a2w = A[:, 4 * C:9 * C]
    pv = a1w * vvgw
    pt = a2w * tgw
    p00 = a0 * s
    p11 = (pv[:, 0:C] + pv[:, C:2 * C] + pv[:, 2 * C:3 * C]) * INV_SQRT3
    p22 = (pt[:, 0:C] + pt[:, C:2 * C] + pt[:, 2 * C:3 * C]
           + pt[:, 3 * C:4 * C] + pt[:, 4 * C:5 * C]) * INV_SQRT5
    xl = jnp.concatenate([x1, p00, p11, p22], axis=1)
    h = _silu(_dot(xl, wlat1_ref[...]))
    yz = _dot(h, wlat2e_ref[...])
    x2 = (x1 + (0.5 * env) * yz[:, 0:H]) * INV_SQRT125
    x2_ref[...] = x2
    wew = (x1we + (0.5 * env) * yz[:, H:H + 9 * C]) * INV_SQRT125
    wy = ye * wew
    wy_ref[...] = jnp.concatenate(
        [wy, jnp.zeros((wy.shape[0], W9 - 9 * C), jnp.float32)], axis=1)
    ts = _dot(jnp.concatenate([a0, s], axis=1), tmat_ref[...])
    a0_3 = ts[:, 0:3 * C]
    a0_5 = ts[:, 3 * C:8 * C]
    s3 = ts[:, 8 * C:11 * C]
    s5 = ts[:, 11 * C:16 * C]
    ax, ay, az = a1w[:, 0:C], a1w[:, C:2 * C], a1w[:, 2 * C:3 * C]
    bx, by, bz = vvgw[:, 0:C], vvgw[:, C:2 * C], vvgw[:, 2 * C:3 * C]
    r11w = jnp.concatenate(
        [S2 * (ax * by + ay * bx),
         S2 * (ay * bz + az * by),
         S6 * (2.0 * az * bz - ax * bx - ay * by),
         S2 * (ax * bz + az * bx),
         S2 * (ax * bx - ay * by)], axis=1)
    xall = jnp.concatenate(
        [a0_3 * vvgw, a1w * s3, a0_5 * tgw, a2w * s5, r11w], axis=1)
    vvt_ref[...] = _dot(xall, wall_ref[...])


def _pass3_body(x2_ref, a_ref, vvt_ref, geom_ref, wlat1r_ref, fo_ref,
                go_ref, o_ref):
    x2 = x2_ref[...]
    env = geom_ref[:, 0:1]
    A = a_ref[...][:, 0:9 * C] * INV_SQRT_ANN
    vvt = vvt_ref[...]
    pv = A[:, C:4 * C] * vvt[:, 0:3 * C]
    pt = A[:, 4 * C:9 * C] * vvt[:, 3 * C:8 * C]
    p11 = (pv[:, 0:C] + pv[:, C:2 * C] + pv[:, 2 * C:3 * C]) * INV_SQRT3
    p22 = (pt[:, 0:C] + pt[:, C:2 * C] + pt[:, 2 * C:3 * C]
           + pt[:, 3 * C:4 * C] + pt[:, 4 * C:5 * C]) * INV_SQRT5
    xl = jnp.concatenate([x2, p11, p22], axis=1)
    h = _silu(_dot(xl, wlat1r_ref[...]))
    o = (jnp.sum(x2 * fo_ref[...], axis=1, keepdims=True)
         + (0.5 * env) * jnp.sum(h * go_ref[...], axis=1, keepdims=True))
    o_ref[...] = o * INV_SQRT125


def _full(shape):
    return pl.BlockSpec(shape, lambda i: (0, 0))


def _blk(shape):
    return pl.BlockSpec(shape, lambda i: (i, 0))


BN = 400


def _node_proj(na, w1b, w1c):
    return pl.pallas_call(
        _node_proj_body,
        grid=(N_NODES // BN,),
        in_specs=[_blk((BN, D_FEAT)), _full((D_FEAT, 16)), _full((D_FEAT, 16))],
        out_specs=[_blk((BN, 16)), _blk((BN, 16))],
        out_shape=[jax.ShapeDtypeStruct((N_NODES, 16), _F32)] * 2,
    )(na, w1b, w1c)


def _pass1(vec, gs, gr, w1a, we2, we3, we4, wenv9, rmat):
    return pl.pallas_call(
        _pass1_body,
        grid=(N_BLK,),
        in_specs=[_blk((BE, 3)), _blk((BE, 16)), _blk((BE, 16)),
                  _full((N_BASIS + 1, 16)), _full((16, 32)), _full((32, 64)),
                  _full((64, 128)), _full((H, 9 * C)), _full((9, 9 * C))],
        out_specs=[_blk((BE, H)), _blk((BE, W9)), _blk((BE, 16))],
        out_shape=[jax.ShapeDtypeStruct((EP, H), _F32),
                   jax.ShapeDtypeStruct((EP, W9), _F32),
                   jax.ShapeDtypeStruct((EP, 16), _F32)],
    )(vec, gs, gr, w1a, we2, we3, we4, wenv9, rmat)


def _pass2(x1, a0, geom, wcomb, wlat1, wlat2e, wall, rmat, tmat):
    return pl.pallas_call(
        _pass2_body,
        grid=(N_BLK,),
        in_specs=[_blk((BE, H)), _blk((BE, W9)), _blk((BE, 16)),
                  _full((H, 18 * C)), _full((H + 3 * C, H)),
                  _full((H, H + 9 * C)), _full((21 * C, 8 * C)),
                  _full((9, 9 * C)), _full((2 * C, 16 * C))],
        out_specs=[_blk((BE, H)), _blk((BE, W9)), _blk((BE, 8 * C))],
        out_shape=[jax.ShapeDtypeStruct((EP, H), _F32),
                   jax.ShapeDtypeStruct((EP, W9), _F32),
                   jax.ShapeDtypeStruct((EP, 8 * C), _F32)],
    )(x1, a0, geom, wcomb, wlat1, wlat2e, wall, rmat, tmat)


def _pass3(x2, a1, vvt, geom, wlat1r, fo, go):
    return pl.pallas_call(
        _pass3_body,
        grid=(N_BLK,),
        in_specs=[_blk((BE, H)), _blk((BE, W9)), _blk((BE, 8 * C)),
                  _blk((BE, 16)), _full((H + 2 * C, H)), _full((1, H)),
                  _full((1, H))],
        out_specs=_blk((BE, 1)),
        out_shape=jax.ShapeDtypeStruct((EP, 1), _F32),
    )(x2, a1, vvt, geom, wlat1r, fo, go)


NW = 32
CH = 128
EPW = EP // NW
EPT = EP // 16
W9 = 3 * H
NHALF = N_NODES // 2
NSH = 5120
NROWT = NSH // 16


@functools.cache
def _sc_kernels():
    mesh = plsc.VectorSubcoreMesh(core_axis_name="c", subcore_axis_name="s",
                                  num_cores=2, num_subcores=16)
    params = pltpu.CompilerParams(use_tc_tiling_on_sc=False)

    @functools.partial(
        pl.kernel,
        out_type=[jax.ShapeDtypeStruct((EP, 16), _F32),
                  jax.ShapeDtypeStruct((EP, 16), _F32)],
        mesh=mesh,
        compiler_params=params,
        scratch_types=[
            pltpu.VMEM((CH,), jnp.int32),
            pltpu.VMEM((CH,), jnp.int32),
            pltpu.VMEM((CH, 16), _F32),
            pltpu.VMEM((CH, 16), _F32),
            pltpu.SemaphoreType.DMA,
            pltpu.SemaphoreType.DMA,
        ],
    )
    def sc_gather(ps_hbm, pr_hbm, snd_hbm, rcv_hbm, outs_hbm, outr_hbm,
                  idxs_v, idxr_v, rows_s, rows_r, sem_s, sem_r):
        wid = lax.axis_index("s") * 2 + lax.axis_index("c")
        base = wid * EPW

        @pl.loop(0, EPW // CH)
        def _(i):
            e0 = base + i * CH
            pltpu.sync_copy(snd_hbm.at[pl.ds(e0, CH)], idxs_v)
            pltpu.sync_copy(rcv_hbm.at[pl.ds(e0, CH)], idxr_v)
            cs = pltpu.async_copy(ps_hbm.at[idxs_v], rows_s, sem_s)
            cr = pltpu.async_copy(pr_hbm.at[idxr_v], rows_r, sem_r)
            cs.wait()
            cr.wait()
            pltpu.sync_copy(rows_s, outs_hbm.at[pl.ds(e0, CH)])
            pltpu.sync_copy(rows_r, outr_hbm.at[pl.ds(e0, CH)])

    tiled = pltpu.CompilerParams(use_tc_tiling_on_sc=True)

    @functools.partial(
        pl.kernel,
        out_type=jax.ShapeDtypeStruct((2 * NSH, W9), _F32),
        mesh=mesh,
        compiler_params=tiled,
        scratch_types=[
            pltpu.VMEM((8, CH), jnp.int32),
            pltpu.VMEM((2, CH, H), _F32),
            pltpu.VMEM_SHARED((NSH, H), _F32),
            pltpu.SemaphoreType.DMA,
            pltpu.SemaphoreType.DMA,
        ],
    )
    def sc_scatter(wy_hbm, snd3d_hbm, zrows_hbm, tab_hbm, idx_v, rows_v,
                   acc_sh, sem0, sem1):
        cid = lax.axis_index("c")
        sid = lax.axis_index("s")
        lo = cid * NHALF
        base = sid * EPT
        lane = lax.iota(jnp.int32, 16)
        for ph in range(3):
            pltpu.sync_copy(zrows_hbm.at[pl.ds(sid * NROWT, NROWT)],
                            acc_sh.at[pl.ds(sid * NROWT, NROWT)])
            plsc.subcore_barrier()

            @pl.loop(0, EPT // (8 * CH))
            def _(i):
                pltpu.sync_copy(snd3d_hbm.at[base // (8 * CH) + i], idx_v)
                for j in range(8):
                    for t in range(8):
                        v = idx_v[j, pl.ds(16 * t, 16)] - lo
                        oob = (v < 0) | (v >= NHALF)
                        trash = NHALF + ((lane + 16 * t + j) & 63)
                        idx_v[j, pl.ds(16 * t, 16)] = jnp.where(oob, trash, v)
                sems = [sem0, sem1]
                wps = [None, None]
                for j in range(8):
                    e0 = base + i * 8 * CH + j * CH
                    if wps[j % 2] is not None:
                        wps[j % 2].wait()
                    pltpu.sync_copy(
                        wy_hbm.at[pl.ds(e0, CH), pl.ds(ph * H, H)],
                        rows_v.at[j % 2])
                    wps[j % 2] = pltpu.async_copy(
                        rows_v.at[j % 2], acc_sh.at[idx_v.at[j]],
                        sems[j % 2], add=True)
                wps[0].wait()
                wps[1].wait()

            plsc.subcore_barrier()
            pltpu.sync_copy(
                acc_sh.at[pl.ds(sid * NROWT, NROWT)],
                tab_hbm.at[pl.ds(cid * NSH + sid * NROWT, NROWT),
                           pl.ds(ph * H, H)])
            plsc.subcore_barrier()

    @functools.partial(
        pl.kernel,
        out_type=jax.ShapeDtypeStruct((EP, W9), _F32),
        mesh=mesh,
        compiler_params=tiled,
        scratch_types=[
            pltpu.VMEM((8, CH), jnp.int32),
            pltpu.VMEM((2, CH, W9), _F32),
            pltpu.SemaphoreType.DMA,
            pltpu.SemaphoreType.DMA,
            pltpu.SemaphoreType.DMA,
            pltpu.SemaphoreType.DMA,
        ],
    )
    def sc_gatherback(tab_hbm, snd3d_hbm, out_hbm, idx_v, rows_v,
                      gs0, gs1, ws0, ws1):
        wid = lax.axis_index("s") * 2 + lax.axis_index("c")
        base = wid * EPW
        gsems = [gs0, gs1]
        wsems = [ws0, ws1]

        @pl.loop(0, EPW // (8 * CH))
        def _(i):
            pltpu.sync_copy(snd3d_hbm.at[base // (8 * CH) + i], idx_v)
            for j in range(8):
                for t in range(8):
                    v = idx_v[j, pl.ds(16 * t, 16)]
                    idx_v[j, pl.ds(16 * t, 16)] = jnp.where(
                        v >= NHALF, v + (NSH - NHALF), v)
            cg = [None] * 8
            ww = [None] * 8
            cg[0] = pltpu.async_copy(tab_hbm.at[idx_v.at[0]], rows_v.at[0],
                                     gsems[0])
            for j in range(8):
                e0 = base + i * 8 * CH + j * CH
                cg[j].wait()
                ww[j] = pltpu.async_copy(rows_v.at[j % 2],
                                         out_hbm.at[pl.ds(e0, CH)],
                                         wsems[j % 2])
                if j + 1 < 8:
                    if j >= 1:
                        ww[j - 1].wait()
                    cg[j + 1] = pltpu.async_copy(
                        tab_hbm.at[idx_v.at[j + 1]],
                        rows_v.at[(j + 1) % 2], gsems[(j + 1) % 2])
            ww[6].wait()
            ww[7].wait()

    return sc_gather, sc_scatter, sc_gatherback


def kernel(node_attrs, vectors, senders, receivers, W_e1, W_e2, W_e3, W_e4,
           W_w0, W_env_0, W_lat1_0, W_lat2_0, W_l1_0, W_l2_0,
           W_env_1, W_lat1_1, W_lat2_1, W_l1_1, W_l2_1, W_f, W_out):
    senders = senders.astype(jnp.int32)
    receivers = receivers.astype(jnp.int32)
    pad_idx = jnp.arange(PAD, dtype=jnp.int32) * (N_NODES // PAD)
    snd_p = jnp.concatenate([senders, pad_idx])
    rcv_p = jnp.concatenate([receivers, pad_idx])
    vec_p = jnp.concatenate(
        [vectors.astype(_F32), jnp.zeros((PAD, 3), _F32)])

    sc_gather, sc_scatter, sc_gatherback = _sc_kernels()
    ps, pr = _node_proj(node_attrs.astype(_F32),
                        W_e1[N_BASIS:N_BASIS + D_FEAT],
                        W_e1[N_BASIS + D_FEAT:])
    gs, gr = sc_gather(ps, pr, snd_p, rcv_p)
    w1a = jnp.concatenate([
        W_e1[:N_BASIS] * jnp.asarray(INVSIG_NP.T),
        -jnp.asarray(MU_NP * INVSIG_NP) @ W_e1[:N_BASIS],
    ], axis=0)
    rmat = jnp.asarray(R_NP)
    tmat = jnp.asarray(TMAT_NP)
    wenv9 = jnp.tile(W_env_0, (1, 9))
    x1, wy0, geom = _pass1(vec_p, gs, gr, w1a, W_e2, W_e3, W_e4, wenv9, rmat)
    snd3d = snd_p.reshape(EP // (8 * CH), 8, CH)
    zrows = jnp.zeros((NSH, H), _F32)
    a0 = sc_gatherback(sc_scatter(wy0, snd3d, zrows), snd3d)
    wcomb = jnp.concatenate([
        W_w0[:, 0:C], jnp.tile(W_w0[:, C:2 * C], (1, 3)),
        jnp.tile(W_w0[:, 2 * C:3 * C], (1, 5)), jnp.tile(W_env_1, (1, 9)),
    ], axis=1)
    wlat2e = jnp.concatenate(
        [W_lat2_0, jnp.tile(W_lat2_0 @ W_env_1, (1, 9))], axis=1)
    wall = jnp.zeros((21 * C, 8 * C), _F32)
    for k in range(3):
        wall = wall.at[C * k:C * (k + 1), C * k:C * (k + 1)].set(
            W_l1_0[:C])
        wall = wall.at[3 * C + C * k:3 * C + C * (k + 1),
                       C * k:C * (k + 1)].set(W_l1_0[C:])
    for m in range(5):
        wall = wall.at[6 * C + C * m:6 * C + C * (m + 1),
                       3 * C + C * m:3 * C + C * (m + 1)].set(
            W_l2_0[:C])
        wall = wall.at[11 * C + C * m:11 * C + C * (m + 1),
                       3 * C + C * m:3 * C + C * (m + 1)].set(
            W_l2_0[C:2 * C])
        wall = wall.at[16 * C + C * m:16 * C + C * (m + 1),
                       3 * C + C * m:3 * C + C * (m + 1)].set(
            W_l2_0[2 * C:])
    x2, wy1, vvt = _pass2(x1, a0, geom, wcomb, W_lat1_0, wlat2e, wall,
                          rmat, tmat)
    a1 = sc_gatherback(sc_scatter(wy1, snd3d, zrows), snd3d)
    fo = W_f @ W_out[:H]
    go = W_lat2_1 @ fo
    wlat1r = jnp.concatenate([W_lat1_1[:H], W_lat1_1[H + C:]], axis=0)
    out = _pass3(x2, a1, vvt, geom, wlat1r, fo.reshape(1, H), go.reshape(1, H))
    return out[:N_EDGES]


# speedup vs baseline: 13.4217x; 1.0228x over previous
import functools

import numpy as np
import jax
import jax.numpy as jnp
from jax import lax
from jax.experimental import pallas as pl
from jax.experimental.pallas import tpu as pltpu
from jax.experimental.pallas import tpu_sc as plsc

N_NODES = 10000
N_EDGES = 160000
D_FEAT = 128
N_BASIS = 8
C = 32
P = 6
RC = 4.0
ANN = 32.0
H = 128

EP = 163840
BE = 2048
N_BLK = EP // BE
PAD = EP - N_EDGES

_trapz = getattr(np, 'trapezoid', None) or np.trapz


def _bessel_np(r, n):
    k = np.arange(1, n + 1)[None, :]
    r = r[:, None]
    safe = np.where(r == 0.0, 1.0, r)
    return np.sqrt(2.0) * np.where(r == 0.0, k * np.pi, np.sin(k * np.pi * safe) / safe)


_r = np.linspace(0.0, 1.0, 1000)
_b = _bessel_np(_r, N_BASIS)
_MU = _trapz(_b, _r, axis=0)
_SIG = _trapz((_b - _MU) ** 2, _r, axis=0) ** 0.5
MU_NP = np.asarray(_MU, dtype=np.float32)[None, :]
INVSIG_NP = np.asarray(1.0 / _SIG, dtype=np.float32)[None, :]

SQRT2 = float(np.sqrt(2.0))
SQRT3 = float(np.sqrt(3.0))
SQRT5 = float(np.sqrt(5.0))
SQRT15 = float(np.sqrt(15.0))
INV_SQRT3 = float(1.0 / np.sqrt(3.0))
INV_SQRT5 = float(1.0 / np.sqrt(5.0))
INV_SQRT_ANN = float(1.0 / np.sqrt(ANN))
INV_SQRT125 = float(1.0 / np.sqrt(1.25))
S2 = float(1.0 / np.sqrt(2.0))
S6 = float(1.0 / np.sqrt(6.0))

_F32 = jnp.float32

R_NP = np.zeros((9, 9 * C), dtype=np.float32)
for _j in range(9):
    R_NP[_j, C * _j:C * (_j + 1)] = 1.0
TMAT_NP = np.zeros((2 * C, 16 * C), dtype=np.float32)
for _k in range(3):
    TMAT_NP[np.arange(C), C * _k + np.arange(C)] = 1.0
    TMAT_NP[C + np.arange(C), 8 * C + C * _k + np.arange(C)] = 1.0
for _m in range(5):
    TMAT_NP[np.arange(C), 3 * C + C * _m + np.arange(C)] = 1.0
    TMAT_NP[C + np.arange(C), 11 * C + C * _m + np.arange(C)] = 1.0


def _silu(x):
    return x * jax.nn.sigmoid(x)


def _dot(a, b):
    return jnp.dot(a, b, preferred_element_type=jnp.float32)


def _node_proj_body(na_ref, w1b_ref, w1c_ref, ps_ref, pr_ref):
    na = na_ref[...]
    ps_ref[...] = _dot(na, w1b_ref[...])
    pr_ref[...] = _dot(na, w1c_ref[...])


def _pass1_body(vec_ref, gs_ref, gr_ref, w1a_ref, we2_ref, we3_ref, we4_ref,
                wenv9_ref, rmat_ref, x1_ref, wy_ref, geom_ref):
    v = vec_ref[...] * (1.0 / RC)
    d2 = jnp.sum(v * v, axis=1, keepdims=True)
    d = jnp.sqrt(d2)
    iszero = d == 0.0
    safe = jnp.where(iszero, 1.0, d)
    kpi = np.float32(np.pi) * (
        lax.broadcasted_iota(jnp.int32, (1, N_BASIS), 1) + 1).astype(
            jnp.float32)
    rb = SQRT2 * jnp.where(iszero, kpi, jnp.sin(kpi * safe) / safe)
    rb9 = jnp.concatenate([rb, jnp.ones_like(d)], axis=1)
    pre1 = _dot(rb9, w1a_ref[...]) + gs_ref[...] + gr_ref[...]
    pre1 = jnp.where(iszero, 0.0, pre1)
    x = _silu(pre1)
    x = _silu(_dot(x, we2_ref[...]))
    x = _silu(_dot(x, we3_ref[...]))
    x = _dot(x, we4_ref[...])
    d6 = d2 * d2 * d2
    d7 = d6 * d
    d8 = d7 * d
    env = jnp.where(d < 1.0, 1.0 - 28.0 * d6 + 48.0 * d7 - 21.0 * d8, 0.0)
    x = env * x
    rh = v / safe
    xx = rh[:, 0:1]
    yy = rh[:, 1:2]
    zz = rh[:, 2:3]
    Y = jnp.concatenate([
        jnp.ones_like(xx), SQRT3 * xx, SQRT3 * yy, SQRT3 * zz,
        SQRT15 * xx * yy, SQRT15 * yy * zz, (SQRT5 / 2.0) * (3.0 * zz * zz - 1.0),
        SQRT15 * xx * zz, (SQRT15 / 2.0) * (xx * xx - yy * yy)
    ], axis=1)
    x1_ref[...] = x
    ye = _dot(Y, rmat_ref[...])
    wy = ye * _dot(x, wenv9_ref[...])
    wy_ref[...] = jnp.concatenate(
        [wy, jnp.zeros((wy.shape[0], W9 - 9 * C), jnp.float32)], axis=1)
    geom_ref[...] = jnp.concatenate(
        [env, Y, jnp.zeros_like(Y[:, :6])], axis=1)


def _pass2_body(x1_ref, a_ref, geom_ref, wcomb_ref, wlat1_ref, wlat2e_ref,
                wall_ref, rmat_ref, tmat_ref, x2_ref, wy_ref, vvt_ref):
    x1 = x1_ref[...]
    g = geom_ref[...]
    env = g[:, 0:1]
    ye = _dot(g[:, 1:10], rmat_ref[...])
    wfull = _dot(x1, wcomb_ref[...])
    s = wfull[:, 0:C]
    wv3 = wfull[:, C:4 * C]
    wt5 = wfull[:, 4 * C:9 * C]
    x1we = wfull[:, 9 * C:18 * C]
    vvgw = ye[:, C:4 * C] * wv3
    tgw = ye[:, 4 * C:9 * C] * wt5
    A = a_ref[...][:, 0:9 * C] * INV_SQRT_ANN
    a0 = A[:, 0:C]
    a1w = A[:, C:4 * C]
    a2w = A[:, 4 * C:9 * C]
    pv = a1w * vvgw
    pt = a2w * tgw
    p00 = a0 * s
    p11 = (pv[:, 0:C] + pv[:, C:2 * C] + pv[:, 2 * C:3 * C]) * INV_SQRT3
    p22 = (pt[:, 0:C] + pt[:, C:2 * C] + pt[:, 2 * C:3 * C]
           + pt[:, 3 * C:4 * C] + pt[:, 4 * C:5 * C]) * INV_SQRT5
    xl = jnp.concatenate([x1, p00, p11, p22], axis=1)
    h = _silu(_dot(xl, wlat1_ref[...]))
    yz = _dot(h, wlat2e_ref[...])
    x2 = (x1 + (0.5 * env) * yz[:, 0:H]) * INV_SQRT125
    x2_ref[...] = x2
    wew = (x1we + (0.5 * env) * yz[:, H:H + 9 * C]) * INV_SQRT125
    wy = ye * wew
    wy_ref[...] = jnp.concatenate(
        [wy, jnp.zeros((wy.shape[0], W9 - 9 * C), jnp.float32)], axis=1)
    ts = _dot(jnp.concatenate([a0, s], axis=1), tmat_ref[...])
    a0_3 = ts[:, 0:3 * C]
    a0_5 = ts[:, 3 * C:8 * C]
    s3 = ts[:, 8 * C:11 * C]
    s5 = ts[:, 11 * C:16 * C]
    ax, ay, az = a1w[:, 0:C], a1w[:, C:2 * C], a1w[:, 2 * C:3 * C]
    bx, by, bz = vvgw[:, 0:C], vvgw[:, C:2 * C], vvgw[:, 2 * C:3 * C]
    r11w = jnp.concatenate(
        [S2 * (ax * by + ay * bx),
         S2 * (ay * bz + az * by),
         S6 * (2.0 * az * bz - ax * bx - ay * by),
         S2 * (ax * bz + az * bx),
         S2 * (ax * bx - ay * by)], axis=1)
    xall = jnp.concatenate(
        [a0_3 * vvgw, a1w * s3, a0_5 * tgw, a2w * s5, r11w], axis=1)
    vvt_ref[...] = _dot(xall, wall_ref[...])


def _pass3_body(x2_ref, a_ref, vvt_ref, geom_ref, wlat1r_ref, fo_ref,
                go_ref, o_ref):
    x2 = x2_ref[...]
    env = geom_ref[:, 0:1]
    A = a_ref[...][:, 0:9 * C] * INV_SQRT_ANN
    vvt = vvt_ref[...]
    pv = A[:, C:4 * C] * vvt[:, 0:3 * C]
    pt = A[:, 4 * C:9 * C] * vvt[:, 3 * C:8 * C]
    p11 = (pv[:, 0:C] + pv[:, C:2 * C] + pv[:, 2 * C:3 * C]) * INV_SQRT3
    p22 = (pt[:, 0:C] + pt[:, C:2 * C] + pt[:, 2 * C:3 * C]
           + pt[:, 3 * C:4 * C] + pt[:, 4 * C:5 * C]) * INV_SQRT5
    xl = jnp.concatenate([x2, p11, p22], axis=1)
    h = _silu(_dot(xl, wlat1r_ref[...]))
    o = (jnp.sum(x2 * fo_ref[...], axis=1, keepdims=True)
         + (0.5 * env) * jnp.sum(h * go_ref[...], axis=1, keepdims=True))
    o_ref[...] = o * INV_SQRT125


def _full(shape):
    return pl.BlockSpec(shape, lambda i: (0, 0))


def _blk(shape):
    return pl.BlockSpec(shape, lambda i: (i, 0))


BN = 400


def _node_proj(na, w1b, w1c):
    return pl.pallas_call(
        _node_proj_body,
        grid=(N_NODES // BN,),
        in_specs=[_blk((BN, D_FEAT)), _full((D_FEAT, 16)), _full((D_FEAT, 16))],
        out_specs=[_blk((BN, 16)), _blk((BN, 16))],
        out_shape=[jax.ShapeDtypeStruct((N_NODES, 16), _F32)] * 2,
    )(na, w1b, w1c)


def _pass1(vec, gs, gr, w1a, we2, we3, we4, wenv9, rmat):
    return pl.pallas_call(
        _pass1_body,
        grid=(N_BLK,),
        in_specs=[_blk((BE, 3)), _blk((BE, 16)), _blk((BE, 16)),
                  _full((N_BASIS + 1, 16)), _full((16, 32)), _full((32, 64)),
                  _full((64, 128)), _full((H, 9 * C)), _full((9, 9 * C))],
        out_specs=[_blk((BE, H)), _blk((BE, W9)), _blk((BE, 16))],
        out_shape=[jax.ShapeDtypeStruct((EP, H), _F32),
                   jax.ShapeDtypeStruct((EP, W9), _F32),
                   jax.ShapeDtypeStruct((EP, 16), _F32)],
    )(vec, gs, gr, w1a, we2, we3, we4, wenv9, rmat)


def _pass2(x1, a0, geom, wcomb, wlat1, wlat2e, wall, rmat, tmat):
    return pl.pallas_call(
        _pass2_body,
        grid=(N_BLK,),
        in_specs=[_blk((BE, H)), _blk((BE, W9)), _blk((BE, 16)),
                  _full((H, 18 * C)), _full((H + 3 * C, H)),
                  _full((H, H + 9 * C)), _full((21 * C, 8 * C)),
                  _full((9, 9 * C)), _full((2 * C, 16 * C))],
        out_specs=[_blk((BE, H)), _blk((BE, W9)), _blk((BE, 8 * C))],
        out_shape=[jax.ShapeDtypeStruct((EP, H), _F32),
                   jax.ShapeDtypeStruct((EP, W9), _F32),
                   jax.ShapeDtypeStruct((EP, 8 * C), _F32)],
    )(x1, a0, geom, wcomb, wlat1, wlat2e, wall, rmat, tmat)


def _pass3(x2, a1, vvt, geom, wlat1r, fo, go):
    return pl.pallas_call(
        _pass3_body,
        grid=(N_BLK,),
        in_specs=[_blk((BE, H)), _blk((BE, W9)), _blk((BE, 8 * C)),
                  _blk((BE, 16)), _full((H + 2 * C, H)), _full((1, H)),
                  _full((1, H))],
        out_specs=_blk((BE, 1)),
        out_shape=jax.ShapeDtypeStruct((EP, 1), _F32),
    )(x2, a1, vvt, geom, wlat1r, fo, go)


NW = 32
CH = 128
EPW = EP // NW
EPT = EP // 16
W9 = 3 * H
NHALF = N_NODES // 2
NSH = 5120
NROWT = NSH // 16


@functools.cache
def _sc_kernels():
    mesh = plsc.VectorSubcoreMesh(core_axis_name="c", subcore_axis_name="s",
                                  num_cores=2, num_subcores=16)
    params = pltpu.CompilerParams(use_tc_tiling_on_sc=False)

    @functools.partial(
        pl.kernel,
        out_type=[jax.ShapeDtypeStruct((EP, 16), _F32),
                  jax.ShapeDtypeStruct((EP, 16), _F32)],
        mesh=mesh,
        compiler_params=params,
        scratch_types=[
            pltpu.VMEM((CH,), jnp.int32),
            pltpu.VMEM((CH,), jnp.int32),
            pltpu.VMEM((CH, 16), _F32),
            pltpu.VMEM((CH, 16), _F32),
            pltpu.SemaphoreType.DMA,
            pltpu.SemaphoreType.DMA,
        ],
    )
    def sc_gather(ps_hbm, pr_hbm, snd_hbm, rcv_hbm, outs_hbm, outr_hbm,
                  idxs_v, idxr_v, rows_s, rows_r, sem_s, sem_r):
        wid = lax.axis_index("s") * 2 + lax.axis_index("c")
        base = wid * EPW

        @pl.loop(0, EPW // CH)
        def _(i):
            e0 = base + i * CH
            pltpu.sync_copy(snd_hbm.at[pl.ds(e0, CH)], idxs_v)
            pltpu.sync_copy(rcv_hbm.at[pl.ds(e0, CH)], idxr_v)
            cs = pltpu.async_copy(ps_hbm.at[idxs_v], rows_s, sem_s)
            cr = pltpu.async_copy(pr_hbm.at[idxr_v], rows_r, sem_r)
            cs.wait()
            cr.wait()
            pltpu.sync_copy(rows_s, outs_hbm.at[pl.ds(e0, CH)])
            pltpu.sync_copy(rows_r, outr_hbm.at[pl.ds(e0, CH)])

    tiled = pltpu.CompilerParams(use_tc_tiling_on_sc=True)

    @functools.partial(
        pl.kernel,
        out_type=jax.ShapeDtypeStruct((2 * NSH, W9), _F32),
        mesh=mesh,
        compiler_params=tiled,
        scratch_types=[
            pltpu.VMEM((8, CH), jnp.int32),
            pltpu.VMEM((2, CH, H), _F32),
            pltpu.VMEM_SHARED((NSH, H), _F32),
            pltpu.SemaphoreType.DMA,
            pltpu.SemaphoreType.DMA,
        ],
    )
    def sc_scatter(wy_hbm, snd3d_hbm, zrows_hbm, tab_hbm, idx_v, rows_v,
                   acc_sh, sem0, sem1):
        cid = lax.axis_index("c")
        sid = lax.axis_index("s")
        lo = cid * NHALF
        base = sid * EPT
        lane = lax.iota(jnp.int32, 16)
        for ph in range(3):
            pltpu.sync_copy(zrows_hbm.at[pl.ds(sid * NROWT, NROWT)],
                            acc_sh.at[pl.ds(sid * NROWT, NROWT)])
            plsc.subcore_barrier()

            @pl.loop(0, EPT // (8 * CH))
            def _(i):
                pltpu.sync_copy(snd3d_hbm.at[base // (8 * CH) + i], idx_v)
                for j in range(8):
                    for t in range(8):
                        v = idx_v[j, pl.ds(16 * t, 16)] - lo
                        oob = (v < 0) | (v >= NHALF)
                        trash = NHALF + ((lane + 16 * t + j) & 63)
                        idx_v[j, pl.ds(16 * t, 16)] = jnp.where(oob, trash, v)
                sems = [sem0, sem1]
                wps = [None, None]
                for j in range(8):
                    e0 = base + i * 8 * CH + j * CH
                    if wps[j % 2] is not None:
                        wps[j % 2].wait()
                    pltpu.sync_copy(
                        wy_hbm.at[pl.ds(e0, CH), pl.ds(ph * H, H)],
                        rows_v.at[j % 2])
                    wps[j % 2] = pltpu.async_copy(
                        rows_v.at[j % 2], acc_sh.at[idx_v.at[j]],
                        sems[j % 2], add=True)
                wps[0].wait()
                wps[1].wait()

            plsc.subcore_barrier()
            pltpu.sync_copy(
                acc_sh.at[pl.ds(sid * NROWT, NROWT)],
                tab_hbm.at[pl.ds(cid * NSH + sid * NROWT, NROWT),
                           pl.ds(ph * H, H)])
            plsc.subcore_barrier()

    @functools.partial(
        pl.kernel,
        out_type=jax.ShapeDtypeStruct((EP, W9), _F32),
        mesh=mesh,
        compiler_params=tiled,
        scratch_types=[
            pltpu.VMEM((8, CH), jnp.int32),
            pltpu.VMEM((2, CH, W9), _F32),
            pltpu.SemaphoreType.DMA,
            pltpu.SemaphoreType.DMA,
            pltpu.SemaphoreType.DMA,
            pltpu.SemaphoreType.DMA,
        ],
    )
    def sc_gatherback(tab_hbm, snd3d_hbm, out_hbm, idx_v, rows_v,
                      gs0, gs1, ws0, ws1):
        wid = lax.axis_index("s") * 2 + lax.axis_index("c")
        base = wid * EPW
        gsems = [gs0, gs1]
        wsems = [ws0, ws1]

        @pl.loop(0, EPW // (8 * CH))
        def _(i):
            pltpu.sync_copy(snd3d_hbm.at[base // (8 * CH) + i], idx_v)
            for j in range(8):
                for t in range(8):
                    v = idx_v[j, pl.ds(16 * t, 16)]
                    idx_v[j, pl.ds(16 * t, 16)] = jnp.where(
                        v >= NHALF, v + (NSH - NHALF), v)
            cg = [None] * 8
            ww = [None] * 8
            cg[0] = pltpu.async_copy(tab_hbm.at[idx_v.at[0]], rows_v.at[0],
                                     gsems[0])
            for j in range(8):
                e0 = base + i * 8 * CH + j * CH
                cg[j].wait()
                ww[j] = pltpu.async_copy(rows_v.at[j % 2],
                                         out_hbm.at[pl.ds(e0, CH)],
                                         wsems[j % 2])
                if j + 1 < 8:
                    if j >= 1:
                        ww[j - 1].wait()
                    cg[j + 1] = pltpu.async_copy(
                        tab_hbm.at[idx_v.at[j + 1]],
                        rows_v.at[(j + 1) % 2], gsems[(j + 1) % 2])
            ww[6].wait()
            ww[7].wait()

    return sc_gather, sc_scatter, sc_gatherback


def kernel(node_attrs, vectors, senders, receivers, W_e1, W_e2, W_e3, W_e4,
           W_w0, W_env_0, W_lat1_0, W_lat2_0, W_l1_0, W_l2_0,
           W_env_1, W_lat1_1, W_lat2_1, W_l1_1, W_l2_1, W_f, W_out):
    senders = senders.astype(jnp.int32)
    receivers = receivers.astype(jnp.int32)
    pad_idx = jnp.arange(PAD, dtype=jnp.int32) * (N_NODES // PAD)
    snd_p = jnp.concatenate([senders, pad_idx])
    rcv_p = jnp.concatenate([receivers, pad_idx])
    vec_p = jnp.concatenate(
        [vectors.astype(_F32), jnp.zeros((PAD, 3), _F32)])

    sc_gather, sc_scatter, sc_gatherback = _sc_kernels()
    ps, pr = _node_proj(node_attrs.astype(_F32),
                        W_e1[N_BASIS:N_BASIS + D_FEAT],
                        W_e1[N_BASIS + D_FEAT:])
    gs, gr = sc_gather(ps, pr, snd_p, rcv_p)
    w1a = jnp.concatenate([
        W_e1[:N_BASIS] * jnp.asarray(INVSIG_NP.T),
        -jnp.asarray(MU_NP * INVSIG_NP) @ W_e1[:N_BASIS],
    ], axis=0)
    rmat = jnp.asarray(R_NP)
    tmat = jnp.asarray(TMAT_NP)
    wenv9 = jnp.tile(W_env_0, (1, 9))
    x1, wy0, geom = _pass1(vec_p, gs, gr, w1a, W_e2, W_e3, W_e4, wenv9, rmat)
    snd3d = snd_p.reshape(EP // (8 * CH), 8, CH)
    zrows = jnp.zeros((NSH, H), _F32)
    a0 = sc_gatherback(sc_scatter(wy0, snd3d, zrows), snd3d)
    wcomb = jnp.concatenate([
        W_w0[:, 0:C], jnp.tile(W_w0[:, C:2 * C], (1, 3)),
        jnp.tile(W_w0[:, 2 * C:3 * C], (1, 5)), jnp.tile(W_env_1, (1, 9)),
    ], axis=1)
    wlat2e = jnp.concatenate(
        [W_lat2_0, jnp.tile(W_lat2_0 @ W_env_1, (1, 9))], axis=1)
    wall = jnp.zeros((21 * C, 8 * C), _F32)
    for k in range(3):
        wall = wall.at[C * k:C * (k + 1), C * k:C * (k + 1)].set(
            W_l1_0[:C])
        wall = wall.at[3 * C + C * k:3 * C + C * (k + 1),
                       C * k:C * (k + 1)].set(W_l1_0[C:])
    for m in range(5):
        wall = wall.at[6 * C + C * m:6 * C + C * (m + 1),
                       3 * C + C * m:3 * C + C * (m + 1)].set(
            W_l2_0[:C])
        wall = wall.at[11 * C + C * m:11 * C + C * (m + 1),
                       3 * C + C * m:3 * C + C * (m + 1)].set(
            W_l2_0[C:2 * C])
        wall = wall.at[16 * C + C * m:16 * C + C * (m + 1),
                       3 * C + C * m:3 * C + C * (m + 1)].set(
            W_l2_0[2 * C:])
    x2, wy1, vvt = _pass2(x1, a0, geom, wcomb, W_lat1_0, wlat2e, wall,
                          rmat, tmat)
    a1 = sc_gatherback(sc_scatter(wy1, snd3d, zrows), snd3d)
    fo = W_f @ W_out[:H]
    go = W_lat2_1 @ fo
    wlat1r = jnp.concatenate([W_lat1_1[:H], W_lat1_1[H + C:]], axis=0)
    out = _pass3(x2, a1, vvt, geom, wlat1r, fo.reshape(1, H), go.reshape(1, H))
    return out[:N_EDGES]
